# Initial kernel scaffold; baseline (speedup 1.0000x reference)
import jax
import jax.numpy as jnp
from jax.experimental import pallas as pl


def kernel(x, edge_index, edge_attr, edge_nei, batch, params):
    raise NotImplementedError("write your pallas kernel here")


# speedup vs baseline: 1.4511x; 1.4511x over previous
import functools
import math

import jax
import jax.numpy as jnp
from jax import lax
from jax.experimental import pallas as pl
from jax.experimental.pallas import tpu as pltpu
from jax.experimental.pallas import tpu_sc as plsc

N_NODES = 10000
N_EDGES = 160000
N_GRAPHS = 64
EMB = 128
BINS = 512

RE = 640
RN = 1000
NC, NS = 2, 16
NW = NC * NS
CHUNK = 128
CPW = 40
TOTAL_CHUNKS = N_EDGES // CHUNK
EPAD = NW * CPW * CHUNK
ROWS_PER_TILE = N_NODES // NS
ISQ = 1.0 / math.sqrt(EMB)

_f32 = jnp.float32


def _tc(body, grid, in_specs, out_specs, out_shape, name):
    return pl.pallas_call(
        body,
        grid=grid,
        in_specs=in_specs,
        out_specs=out_specs,
        out_shape=out_shape,
        compiler_params=pltpu.CompilerParams(
            dimension_semantics=("arbitrary",)),
        name=name,
    )


def _full(shape):
    return pl.BlockSpec(shape, lambda i: (0,) * len(shape))


def _rows(r, cols):
    return pl.BlockSpec((r, cols), lambda i: (i, 0))


def _embed_tables(x, wemb, bemb, wd, bd, ws):
    def body(x_r, wemb_r, bemb_r, wd_r, bd_r, ws_r, node_r, td_r, ts_r):
        node = jnp.dot(x_r[...], wemb_r[...], preferred_element_type=_f32)
        node = node + bemb_r[...]
        node_r[...] = node
        td_r[...] = jnp.dot(node, wd_r[...], preferred_element_type=_f32) + bd_r[...]
        ts_r[...] = jnp.dot(node, ws_r[...], preferred_element_type=_f32)

    return _tc(
        body, (N_NODES // RN,),
        [_rows(RN, 92), _full((92, EMB)), _full((1, EMB)),
         _full((EMB, 3 * EMB)), _full((1, 3 * EMB)), _full((EMB, 2 * EMB))],
        [_rows(RN, EMB), _rows(RN, 3 * EMB), _rows(RN, 2 * EMB)],
        [jax.ShapeDtypeStruct((N_NODES, EMB), _f32),
         jax.ShapeDtypeStruct((N_NODES, 3 * EMB), _f32),
         jax.ShapeDtypeStruct((N_NODES, 2 * EMB), _f32)],
        "embed_tables",
    )(x, wemb, bemb, wd, bd, ws)


def _node_update(node_prev, outlin, sums, wd, bd, ws, make_tables):
    def body(np_r, ol_r, sums_r, wd_r, bd_r, ws_r, node_r, td_r, ts_r):
        s = sums_r[...]
        m = s[0:1, :] / N_NODES
        v = s[1:2, :] / N_NODES - m * m
        node = jax.nn.softplus(np_r[...] + (ol_r[...] - m) / jnp.sqrt(v + 1e-5))
        node_r[...] = node
        td_r[...] = jnp.dot(node, wd_r[...], preferred_element_type=_f32) + bd_r[...]
        ts_r[...] = jnp.dot(node, ws_r[...], preferred_element_type=_f32)

    def body_plain(np_r, ol_r, sums_r, node_r):
        s = sums_r[...]
        m = s[0:1, :] / N_NODES
        v = s[1:2, :] / N_NODES - m * m
        node_r[...] = jax.nn.softplus(
            np_r[...] + (ol_r[...] - m) / jnp.sqrt(v + 1e-5))

    if make_tables:
        return _tc(
            body, (N_NODES // RN,),
            [_rows(RN, EMB), _rows(RN, EMB), _full((2, EMB)),
             _full((EMB, 3 * EMB)), _full((1, 3 * EMB)), _full((EMB, 2 * EMB))],
            [_rows(RN, EMB), _rows(RN, 3 * EMB), _rows(RN, 2 * EMB)],
            [jax.ShapeDtypeStruct((N_NODES, EMB), _f32),
             jax.ShapeDtypeStruct((N_NODES, 3 * EMB), _f32),
             jax.ShapeDtypeStruct((N_NODES, 2 * EMB), _f32)],
            "node_update_tables",
        )(node_prev, outlin, sums, wd, bd, ws)
    return _tc(
        body_plain, (N_NODES // RN,),
        [_rows(RN, EMB), _rows(RN, EMB), _full((2, EMB))],
        [_rows(RN, EMB)],
        [jax.ShapeDtypeStruct((N_NODES, EMB), _f32)],
        "node_update",
    )(node_prev, outlin, sums)[0]


def _rbf_from_norm(vecs, w, b, vmin, vmax):
    L = vecs.shape[0]
    gamma = (BINS - 1) / (vmax - vmin)
    step = (vmax - vmin) / (BINS - 1)

    def body(v_r, w_r, b_r, o_r):
        v = v_r[...]
        d = -0.75 / jnp.sqrt(jnp.sum(v * v, axis=1, keepdims=True))
        c = vmin + step * lax.broadcasted_iota(
            jnp.int32, (RE, BINS), 1).astype(_f32)
        ex = jnp.exp(-gamma * (d - c) ** 2)
        o_r[...] = jax.nn.softplus(
            jnp.dot(ex, w_r[...], preferred_element_type=_f32) + b_r[...])

    return _tc(
        body, (L // RE,),
        [_rows(RE, 3), _full((BINS, EMB)), _full((1, EMB))],
        [_rows(RE, EMB)],
        [jax.ShapeDtypeStruct((L, EMB), _f32)],
        "rbf_norm",
    )(vecs, w, b)[0]


def _rbf_from_cos(nei, att, w, b):
    L = nei.shape[0]
    vmin, vmax = -1.0, 1.0
    gamma = (BINS - 1) / (vmax - vmin)
    step = (vmax - vmin) / (BINS - 1)

    def body(n_r, a_r, w_r, b_r, o_r):
        n = n_r[...]
        a = a_r[...]
        nn = jnp.sqrt(jnp.sum(n * n, axis=1, keepdims=True))
        na = jnp.sqrt(jnp.sum(a * a, axis=1, keepdims=True))
        cos = jnp.sum(n * a, axis=1, keepdims=True) / (nn * na)
        cos = jnp.clip(cos, -1.0, 1.0)
        c = vmin + step * lax.broadcasted_iota(
            jnp.int32, (RE, BINS), 1).astype(_f32)
        ex = jnp.exp(-gamma * (cos - c) ** 2)
        o_r[...] = jax.nn.softplus(
            jnp.dot(ex, w_r[...], preferred_element_type=_f32) + b_r[...])

    return _tc(
        body, (L // RE,),
        [_rows(RE, 3), _rows(RE, 3), _full((BINS, EMB)), _full((1, EMB))],
        [_rows(RE, EMB)],
        [jax.ShapeDtypeStruct((L, EMB), _f32)],
        "rbf_cos",
    )(nei, att, w, b)[0]


def _conv_phase1(gd, gs, ef_args, wek, k1, wem, m1, cvec, pre_bn):
    nsteps = N_EDGES // RE

    def compute(gd_r, gs_r, ef, wek_r, k1_r, wem_r, m1_r, cvec_r,
                alpha_r, msg_r, sums_r):
        c = cvec_r[...]
        g = gd_r[...]
        h = gs_r[...]
        qd = g[:, 0:EMB]
        kad = g[:, EMB:2 * EMB]
        vad = g[:, 2 * EMB:3 * EMB]
        kas = h[:, 0:EMB]
        vas = h[:, EMB:2 * EMB]
        hk = jax.nn.silu(kad + kas +
                         jnp.dot(ef, wek_r[...], preferred_element_type=_f32)
                         + c[0:1, :])
        key_j = jnp.dot(hk, k1_r[...], preferred_element_type=_f32) + c[1:2, :]
        alpha = qd * key_j * ISQ
        hm = jax.nn.silu(vad + vas +
                         jnp.dot(ef, wem_r[...], preferred_element_type=_f32)
                         + c[2:3, :])
        msg = jnp.dot(hm, m1_r[...], preferred_element_type=_f32) + c[3:4, :]
        alpha_r[...] = alpha
        msg_r[...] = msg

        @pl.when(pl.program_id(0) == 0)
        def _():
            sums_r[...] = jnp.zeros_like(sums_r)

        part = jnp.concatenate(
            [jnp.sum(alpha, axis=0, keepdims=True),
             jnp.sum(alpha * alpha, axis=0, keepdims=True)], axis=0)
        sums_r[...] = sums_r[...] + part

    out_specs = [_rows(RE, EMB), _rows(RE, EMB), _full((2, EMB))]
    out_shape = [jax.ShapeDtypeStruct((N_EDGES, EMB), _f32),
                 jax.ShapeDtypeStruct((N_EDGES, EMB), _f32),
                 jax.ShapeDtypeStruct((2, EMB), _f32)]

    if not pre_bn:
        (ef,) = ef_args

        def body(gd_r, gs_r, ef_r, wek_r, k1_r, wem_r, m1_r, cvec_r,
                 alpha_r, msg_r, sums_r):
            compute(gd_r, gs_r, ef_r[...], wek_r, k1_r, wem_r, m1_r, cvec_r,
                    alpha_r, msg_r, sums_r)

        return _tc(
            body, (nsteps,),
            [_rows(RE, 3 * EMB), _rows(RE, 2 * EMB), _rows(RE, EMB),
             _full((EMB, EMB)), _full((EMB, EMB)), _full((EMB, EMB)),
             _full((EMB, EMB)), _full((4, EMB))],
            out_specs, out_shape, "conv_phase1",
        )(gd, gs, ef, wek, k1, wem, m1, cvec)

    ef0, outlin_e, esums = ef_args

    def body2(gd_r, gs_r, ef0_r, ol_r, es_r, wek_r, k1_r, wem_r, m1_r,
              cvec_r, alpha_r, msg_r, sums_r):
        s = es_r[...]
        m = s[0:1, :] / N_EDGES
        v = s[1:2, :] / N_EDGES - m * m
        ef = jax.nn.softplus(ef0_r[...] + (ol_r[...] - m) / jnp.sqrt(v + 1e-5))
        compute(gd_r, gs_r, ef, wek_r, k1_r, wem_r, m1_r, cvec_r,
                alpha_r, msg_r, sums_r)

    return _tc(
        body2, (nsteps,),
        [_rows(RE, 3 * EMB), _rows(RE, 2 * EMB), _rows(RE, EMB),
         _rows(RE, EMB), _full((2, EMB)),
         _full((EMB, EMB)), _full((EMB, EMB)), _full((EMB, EMB)),
         _full((EMB, EMB)), _full((4, EMB))],
        out_specs, out_shape, "conv_phase1_bn",
    )(gd, gs, ef0, outlin_e, esums, wek, k1, wem, m1, cvec)


def _conv_gate(alpha, msg, sums):
    def body(a_r, m_r, s_r, o_r):
        s = s_r[...]
        mean = s[0:1, :] / N_EDGES
        var = s[1:2, :] / N_EDGES - mean * mean
        bn = (a_r[...] - mean) / jnp.sqrt(var + 1e-5)
        o_r[...] = m_r[...] * jax.nn.sigmoid(bn)

    return _tc(
        body, (N_EDGES // RE,),
        [_rows(RE, EMB), _rows(RE, EMB), _full((2, EMB))],
        [_rows(RE, EMB)],
        [jax.ShapeDtypeStruct((N_EDGES, EMB), _f32)],
        "conv_gate",
    )(alpha, msg, sums)[0]


def _conv_concate(agg2, wc, bc):
    nsteps = N_NODES // RN

    def body(a_r, b_r, wc_r, bc_r, o_r, sums_r):
        s = a_r[...] + b_r[...]
        o = jnp.dot(s, wc_r[...], preferred_element_type=_f32) + bc_r[...]
        o_r[...] = o

        @pl.when(pl.program_id(0) == 0)
        def _():
            sums_r[...] = jnp.zeros_like(sums_r)

        part = jnp.concatenate(
            [jnp.sum(o, axis=0, keepdims=True),
             jnp.sum(o * o, axis=0, keepdims=True)], axis=0)
        sums_r[...] = sums_r[...] + part

    spec_a = pl.BlockSpec((RN, EMB), lambda i: (i, 0))
    spec_b = pl.BlockSpec((RN, EMB), lambda i: (i + N_NODES // RN, 0))
    return _tc(
        body, (nsteps,),
        [spec_a, spec_b, _full((EMB, EMB)), _full((1, EMB))],
        [_rows(RN, EMB), _full((2, EMB))],
        [jax.ShapeDtypeStruct((N_NODES, EMB), _f32),
         jax.ShapeDtypeStruct((2, EMB), _f32)],
        "conv_concate",
    )(agg2, agg2, wc, bc)


def _edge_phase_a(ef, nl, na, wq, akx, av, bks, bvs, ck, cv, k1, m1,
                  cks, cvs, oth):
    nsteps = N_EDGES // RE
    nblocks = nsteps

    def body(ef_r, nl0, nl1, nl2, na0, na1, na2, wq_r, akx_r, av_r,
             bks_r, bvs_r, ck_r, cv_r, k1_r, m1_r, cks_r, cvs_r, oth_r,
             a0, a1, a2, v0, v1, v2, sums_r):
        e = ef_r[...]
        othv = oth_r[...]
        q = jnp.dot(e, wq_r[...], preferred_element_type=_f32) + othv[0:1, :]
        ekx = jnp.dot(e, akx_r[...], preferred_element_type=_f32)
        evx = jnp.dot(e, av_r[...], preferred_element_type=_f32)
        k1v = k1_r[...]
        m1v = m1_r[...]
        ckv = ck_r[...]
        cvv = cv_r[...]
        bksv = bks_r[...]
        bvsv = bvs_r[...]
        cksv = cks_r[...]
        cvsv = cvs_r[...]
        nls = (nl0[...], nl1[...], nl2[...])
        nas = (na0[...], na1[...], na2[...])
        aouts = (a0, a1, a2)
        vouts = (v0, v1, v2)
        ssum = jnp.zeros((1, EMB), _f32)
        ssq = jnp.zeros((1, EMB), _f32)
        for t in range(3):
            bk_t = bksv[t * EMB:(t + 1) * EMB, :]
            bv_t = bvsv[t * EMB:(t + 1) * EMB, :]
            hk = jax.nn.silu(
                ekx + jnp.dot(nls[t], bk_t, preferred_element_type=_f32)
                + jnp.dot(nas[t], ckv, preferred_element_type=_f32)
                + cksv[t:t + 1, :])
            kt = jnp.dot(hk, k1v, preferred_element_type=_f32) + othv[1:2, :]
            at = q * kt * ISQ
            aouts[t][...] = at
            hv = jax.nn.silu(
                evx + jnp.dot(nls[t], bv_t, preferred_element_type=_f32)
                + jnp.dot(nas[t], cvv, preferred_element_type=_f32)
                + cvsv[t:t + 1, :])
            vt = jnp.dot(hv, m1v, preferred_element_type=_f32) + othv[2:3, :]
            vouts[t][...] = vt
            ssum = ssum + jnp.sum(at, axis=0, keepdims=True)
            ssq = ssq + jnp.sum(at * at, axis=0, keepdims=True)

        @pl.when(pl.program_id(0) == 0)
        def _():
            sums_r[...] = jnp.zeros_like(sums_r)

        sums_r[...] = sums_r[...] + jnp.concatenate([ssum, ssq], axis=0)

    def tslab(t):
        return pl.BlockSpec((RE, EMB), lambda i, t=t: (t * nblocks + i, 0))

    e_shape = jax.ShapeDtypeStruct((N_EDGES, EMB), _f32)
    return _tc(
        body, (nsteps,),
        [_rows(RE, EMB), tslab(0), tslab(1), tslab(2),
         tslab(0), tslab(1), tslab(2),
         _full((EMB, EMB)), _full((EMB, EMB)), _full((EMB, EMB)),
         _full((3 * EMB, EMB)), _full((3 * EMB, EMB)),
         _full((EMB, EMB)), _full((EMB, EMB)),
         _full((EMB, EMB)), _full((EMB, EMB)),
         _full((3, EMB)), _full((3, EMB)), _full((3, EMB))],
        [_rows(RE, EMB)] * 6 + [_full((2, EMB))],
        [e_shape] * 6 + [jax.ShapeDtypeStruct((2, EMB), _f32)],
        "edge_phase_a",
    )(ef, nl, nl, nl, na, na, na, wq, akx, av, bks, bvs, ck, cv, k1, m1,
      cks, cvs, oth)


def _edge_phase_b(alphas, vals, asums, wc, bc3):
    nsteps = N_EDGES // RE

    def body(a0, a1, a2, v0, v1, v2, as_r, wc_r, bc3_r, o_r, sums_r):
        s = as_r[...]
        mean = s[0:1, :] / (3 * N_EDGES)
        var = s[1:2, :] / (3 * N_EDGES) - mean * mean
        rstd = 1.0 / jnp.sqrt(var + 1e-5)
        acc = jnp.zeros((RE, EMB), _f32)
        for a_r, v_r in ((a0, v0), (a1, v1), (a2, v2)):
            gate = jax.nn.sigmoid((a_r[...] - mean) * rstd)
            acc = acc + v_r[...] * gate
        o = jnp.dot(acc, wc_r[...], preferred_element_type=_f32) + bc3_r[...]
        o_r[...] = o

        @pl.when(pl.program_id(0) == 0)
        def _():
            sums_r[...] = jnp.zeros_like(sums_r)

        part = jnp.concatenate(
            [jnp.sum(o, axis=0, keepdims=True),
             jnp.sum(o * o, axis=0, keepdims=True)], axis=0)
        sums_r[...] = sums_r[...] + part

    return _tc(
        body, (nsteps,),
        [_rows(RE, EMB)] * 6 + [_full((2, EMB)), _full((EMB, EMB)),
                                _full((1, EMB))],
        [_rows(RE, EMB), _full((2, EMB))],
        [jax.ShapeDtypeStruct((N_EDGES, EMB), _f32),
         jax.ShapeDtypeStruct((2, EMB), _f32)],
        "edge_phase_b",
    )(*alphas, *vals, asums, wc, bc3)


def _pool_head(node, batch3, w0, b0, w1, b1, w2, b2):
    nsteps = N_NODES // RN

    def body(n_r, bt_r, w0_r, b0_r, w1_r, b1_r, w2_r, b2_r, o_r,
             sacc, cacc):
        @pl.when(pl.program_id(0) == 0)
        def _():
            sacc[...] = jnp.zeros_like(sacc)
            cacc[...] = jnp.zeros_like(cacc)

        b = bt_r[0]
        oh = (lax.broadcasted_iota(jnp.int32, (N_GRAPHS, RN), 0)
              == b).astype(_f32)
        sacc[...] = sacc[...] + lax.dot_general(
            oh, n_r[...], (((1,), (0,)), ((), ())),
            preferred_element_type=_f32)
        cacc[...] = cacc[...] + jnp.broadcast_to(
            jnp.sum(oh, axis=1, keepdims=True), (N_GRAPHS, EMB))

        @pl.when(pl.program_id(0) == nsteps - 1)
        def _():
            feats = sacc[...] / jnp.maximum(cacc[...], 1.0)
            h = jax.nn.silu(
                jnp.dot(feats, w0_r[...], preferred_element_type=_f32)
                + b0_r[...])
            h = jax.nn.silu(
                jnp.dot(h, w1_r[...], preferred_element_type=_f32)
                + b1_r[...])
            o_r[...] = (jnp.dot(h, w2_r[...], preferred_element_type=_f32)
                        + b2_r[...])

    return pl.pallas_call(
        body,
        grid=(nsteps,),
        in_specs=[_rows(RN, EMB),
                  pl.BlockSpec((1, 1, RN), lambda i: (i, 0, 0)),
                  _full((EMB, EMB)), _full((1, EMB)),
                  _full((EMB, EMB)), _full((1, EMB)),
                  _full((EMB, 6)), _full((1, 6))],
        out_specs=_full((N_GRAPHS, 6)),
        out_shape=jax.ShapeDtypeStruct((N_GRAPHS, 6), _f32),
        scratch_shapes=[pltpu.VMEM((N_GRAPHS, EMB), _f32),
                        pltpu.VMEM((N_GRAPHS, EMB), _f32)],
        compiler_params=pltpu.CompilerParams(
            dimension_semantics=("arbitrary",)),
        name="pool_head",
    )(node, batch3, w0, b0, w1, b1, w2, b2)


def _sc_gather(table, idx_pad, width):
    mesh = plsc.VectorSubcoreMesh(core_axis_name="c", subcore_axis_name="s")

    @functools.partial(
        pl.kernel,
        out_type=jax.ShapeDtypeStruct((N_EDGES, width), _f32),
        mesh=mesh,
        scratch_types=[pltpu.VMEM((CPW * CHUNK,), jnp.int32),
                       pltpu.VMEM((CHUNK, width), _f32),
                       pltpu.SemaphoreType.DMA],
    )
    def k(table_hbm, idx_hbm, out_hbm, idx_v, rows_v, sem):
        w = lax.axis_index("s") * NC + lax.axis_index("c")
        base = w * (CPW * CHUNK)
        pltpu.sync_copy(idx_hbm.at[pl.ds(base, CPW * CHUNK)], idx_v)
        nch = jnp.minimum(CPW, TOTAL_CHUNKS - w * CPW)

        def body(j, carry):
            pltpu.async_copy(
                table_hbm.at[idx_v.at[pl.ds(j * CHUNK, CHUNK)]],
                rows_v, sem).wait()
            pltpu.sync_copy(rows_v,
                            out_hbm.at[pl.ds(base + j * CHUNK, CHUNK)])
            return carry

        lax.fori_loop(0, nch, body, 0)

    return k(table, idx_pad)


def _sc_scatter_add(gated, idx3, zrows):
    mesh = plsc.VectorSubcoreMesh(core_axis_name="c", subcore_axis_name="s")

    NPAD = 10240

    @functools.partial(
        pl.kernel,
        out_type=jax.ShapeDtypeStruct((2 * N_NODES, EMB), _f32),
        mesh=mesh,
        scratch_types=[pltpu.VMEM_SHARED((NPAD, EMB), _f32),
                       pltpu.VMEM((CPW, CHUNK), jnp.int32),
                       pltpu.VMEM((CHUNK, EMB), _f32)],
    )
    def k(g_hbm, idx_hbm, z_hbm, out_hbm, acc, idx_v, chunk_v):
        cid = lax.axis_index("c")
        sid = lax.axis_index("s")
        w = sid * NC + cid
        row0 = sid * 640
        pltpu.sync_copy(z_hbm, acc.at[pl.ds(row0, 640)])
        plsc.subcore_barrier()
        pltpu.sync_copy(idx_hbm.at[w], idx_v)
        base = w * (CPW * CHUNK)
        nch = jnp.minimum(CPW, TOTAL_CHUNKS - w * CPW)

        def body(j, carry):
            pltpu.sync_copy(g_hbm.at[pl.ds(base + j * CHUNK, CHUNK)],
                            chunk_v)
            pltpu.sync_copy(chunk_v, acc.at[idx_v.at[j]], add=True)
            return carry

        lax.fori_loop(0, nch, body, 0)
        plsc.subcore_barrier()

        @pl.when(sid < NS - 1)
        def _():
            pltpu.sync_copy(acc.at[pl.ds(row0, 640)],
                            out_hbm.at[pl.ds(cid * N_NODES + row0, 640)])

        @pl.when(sid == NS - 1)
        def _():
            pltpu.sync_copy(acc.at[pl.ds(row0, 400)],
                            out_hbm.at[pl.ds(cid * N_NODES + row0, 400)])

    return k(gated, idx3, zrows)


def _compose_conv(cp):
    wq, bq = cp["lin_query"]["w"], cp["lin_query"]["b"]
    wk, bk = cp["lin_key"]["w"], cp["lin_key"]["b"]
    wv, bv = cp["lin_value"]["w"], cp["lin_value"]["b"]
    we, be = cp["lin_edge"]["w"], cp["lin_edge"]["b"]
    k0, k0b = cp["key_update"]["l0"]["w"], cp["key_update"]["l0"]["b"]
    k1, k1b = cp["key_update"]["l1"]["w"], cp["key_update"]["l1"]["b"]
    m0, m0b = cp["lin_msg_update"]["l0"]["w"], cp["lin_msg_update"]["l0"]["b"]
    m1, m1b = cp["lin_msg_update"]["l1"]["w"], cp["lin_msg_update"]["l1"]["b"]
    k0i, k0j, k0e = k0[:EMB], k0[EMB:2 * EMB], k0[2 * EMB:]
    m0i, m0j, m0e = m0[:EMB], m0[EMB:2 * EMB], m0[2 * EMB:]
    wd = jnp.concatenate([wq, wk @ k0i, wv @ m0i], axis=1)
    bd = jnp.concatenate(
        [bq, jnp.zeros((EMB,), _f32), jnp.zeros((EMB,), _f32)])[None, :]
    ws = jnp.concatenate([wk @ k0j, wv @ m0j], axis=1)
    wek = we @ k0e
    wem = we @ m0e
    ck = k0b + bk @ k0i + bk @ k0j + be @ k0e
    cm = m0b + bv @ m0i + bv @ m0j + be @ m0e
    cvec = jnp.stack([ck, k1b, cm, m1b], axis=0)
    return dict(wd=wd, bd=bd, ws=ws, wek=wek, wem=wem, k1=k1, m1=m1,
                cvec=cvec, wc=cp["lin_concate"]["w"],
                bc=cp["lin_concate"]["b"][None, :])


def _compose_edge(cp):
    wq, bq = cp["lin_query"]["w"], cp["lin_query"]["b"]
    wk, bk = cp["lin_key"]["w"], cp["lin_key"]["b"]
    wv, bv = cp["lin_value"]["w"], cp["lin_value"]["b"]
    we = cp["lin_edge"]["w"]
    k0, k0b = cp["key_update"]["l0"]["w"], cp["key_update"]["l0"]["b"]
    k1, k1b = cp["key_update"]["l1"]["w"], cp["key_update"]["l1"]["b"]
    m0, m0b = cp["lin_msg_update"]["l0"]["w"], cp["lin_msg_update"]["l0"]["b"]
    m1, m1b = cp["lin_msg_update"]["l1"]["w"], cp["lin_msg_update"]["l1"]["b"]
    k0x, k0y, k0e = k0[:EMB], k0[EMB:2 * EMB], k0[2 * EMB:]
    m0x, m0y, m0e = m0[:EMB], m0[EMB:2 * EMB], m0[2 * EMB:]
    akx = wk @ k0x
    av = wv @ m0x
    ckm = we @ k0e
    cvm = we @ m0e
    bks, bvs, cks, cvs = [], [], [], []
    for t, (ke, ve) in enumerate((("lin_key_e1", "lin_value_e1"),
                                  ("lin_key_e2", "lin_value_e2"),
                                  ("lin_key_e3", "lin_value_e3"))):
        wke, bke = cp[ke]["w"], cp[ke]["b"]
        wve, bve = cp[ve]["w"], cp[ve]["b"]
        bks.append(wke @ k0y)
        bvs.append(wve @ m0y)
        cks.append(k0b + bk @ k0x + bke @ k0y)
        cvs.append(m0b + bv @ m0x + bve @ m0y)
    oth = jnp.stack([bq, k1b, m1b], axis=0)
    return dict(wq=wq, akx=akx, av=av,
                bks=jnp.concatenate(bks, axis=0),
                bvs=jnp.concatenate(bvs, axis=0),
                ck=ckm, cv=cvm, k1=k1, m1=m1,
                cks=jnp.stack(cks, axis=0), cvs=jnp.stack(cvs, axis=0),
                oth=oth, wc=cp["lin_concate"]["w"],
                bc3=3.0 * cp["lin_concate"]["b"][None, :])


def _conv_layer(comp, td, ts, ef_args, src_pad, dst_pad, dst3, zrows,
                node_prev, next_comp):
    gd = _sc_gather(td, dst_pad, 3 * EMB)
    gs = _sc_gather(ts, src_pad, 2 * EMB)
    alpha, msg, asums = _conv_phase1(
        gd, gs, ef_args, comp["wek"], comp["k1"], comp["wem"], comp["m1"],
        comp["cvec"], pre_bn=(len(ef_args) == 3))
    gated = _conv_gate(alpha, msg, asums)
    agg2 = _sc_scatter_add(gated, dst3, zrows)
    outlin, osums = _conv_concate(agg2, comp["wc"], comp["bc"])
    if next_comp is not None:
        return _node_update(node_prev, outlin, osums, next_comp["wd"],
                            next_comp["bd"], next_comp["ws"],
                            make_tables=True)
    return _node_update(node_prev, outlin, osums, None, None, None,
                        make_tables=False)


def kernel(x, edge_index, edge_attr, edge_nei, batch, params):
    src = edge_index[0]
    dst = edge_index[1]
    comp0 = _compose_conv(params["att0"])
    comp1 = _compose_conv(params["att1"])
    compe = _compose_edge(params["edge_update"])
    wrbf, brbf = params["rbf"]["w"], params["rbf"]["b"][None, :]
    wrba, brba = params["rbf_angle"]["w"], params["rbf_angle"]["b"][None, :]

    pad = EPAD - N_EDGES
    src_pad = jnp.pad(src, (0, pad))
    dst_pad = jnp.pad(dst, (0, pad))
    dst3 = dst_pad.reshape(NW, CPW, CHUNK)
    zrows = jnp.zeros((640, EMB), _f32)
    nei_t = jnp.transpose(edge_nei, (1, 0, 2)).reshape(3 * N_EDGES, 3)
    att_rep = jnp.broadcast_to(edge_attr[None, :, :],
                               (3, N_EDGES, 3)).reshape(3 * N_EDGES, 3)
    batch3 = batch.reshape(N_NODES // RN, 1, RN)

    ef0 = _rbf_from_norm(edge_attr, wrbf, brbf, -4.0, 0.0)
    nl = _rbf_from_norm(nei_t, wrbf, brbf, -4.0, 0.0)
    na = _rbf_from_cos(nei_t, att_rep, wrba, brba)

    node0, td0, ts0 = _embed_tables(
        x, params["atom_embedding"]["w"],
        params["atom_embedding"]["b"][None, :],
        comp0["wd"], comp0["bd"], comp0["ws"])

    node1, td1, ts1 = _conv_layer(
        comp0, td0, ts0, (ef0,), src_pad, dst_pad, dst3, zrows,
        node0, comp1)

    alphas_vals = _edge_phase_a(
        ef0, nl, na, compe["wq"], compe["akx"], compe["av"], compe["bks"],
        compe["bvs"], compe["ck"], compe["cv"], compe["k1"], compe["m1"],
        compe["cks"], compe["cvs"], compe["oth"])
    a0, a1, a2, v0, v1, v2, easums = alphas_vals
    outlin_e, esums = _edge_phase_b((a0, a1, a2), (v0, v1, v2), easums,
                                    compe["wc"], compe["bc3"])

    node2 = _conv_layer(
        comp1, td1, ts1, (ef0, outlin_e, esums), src_pad, dst_pad, dst3,
        zrows, node1, None)

    out = _pool_head(node2, batch3,
                     params["fc0"]["w"], params["fc0"]["b"][None, :],
                     params["fc1"]["w"], params["fc1"]["b"][None, :],
                     params["fc_out"]["w"], params["fc_out"]["b"][None, :])
    return jnp.squeeze(out)


# speedup vs baseline: 1.7128x; 1.1804x over previous
import functools
import math

import jax
import jax.numpy as jnp
from jax import lax
from jax.experimental import pallas as pl
from jax.experimental.pallas import tpu as pltpu
from jax.experimental.pallas import tpu_sc as plsc

N_NODES = 10000
N_EDGES = 160000
N_GRAPHS = 64
EMB = 128
BINS = 512

RE = 640
RN = 1000
NC, NS = 2, 16
NW = NC * NS
CHUNK = 128
CPW = 40
TOTAL_CHUNKS = N_EDGES // CHUNK
EPAD = NW * CPW * CHUNK
ROWS_PER_TILE = N_NODES // NS
ISQ = 1.0 / math.sqrt(EMB)

_f32 = jnp.float32


def _tc(body, grid, in_specs, out_specs, out_shape, name):
    return pl.pallas_call(
        body,
        grid=grid,
        in_specs=in_specs,
        out_specs=out_specs,
        out_shape=out_shape,
        compiler_params=pltpu.CompilerParams(
            dimension_semantics=("arbitrary",)),
        name=name,
    )


def _full(shape):
    return pl.BlockSpec(shape, lambda i: (0,) * len(shape))


def _rows(r, cols):
    return pl.BlockSpec((r, cols), lambda i: (i, 0))


def _embed_tables(x, wemb, bemb, wd, bd, ws):
    def body(x_r, wemb_r, bemb_r, wd_r, bd_r, ws_r, node_r, td_r, ts_r):
        node = jnp.dot(x_r[...], wemb_r[...], preferred_element_type=_f32)
        node = node + bemb_r[...]
        node_r[...] = node
        td_r[...] = jnp.dot(node, wd_r[...], preferred_element_type=_f32) + bd_r[...]
        ts_r[...] = jnp.dot(node, ws_r[...], preferred_element_type=_f32)

    return _tc(
        body, (N_NODES // RN,),
        [_rows(RN, 92), _full((92, EMB)), _full((1, EMB)),
         _full((EMB, 3 * EMB)), _full((1, 3 * EMB)), _full((EMB, 2 * EMB))],
        [_rows(RN, EMB), _rows(RN, 3 * EMB), _rows(RN, 2 * EMB)],
        [jax.ShapeDtypeStruct((N_NODES, EMB), _f32),
         jax.ShapeDtypeStruct((N_NODES, 3 * EMB), _f32),
         jax.ShapeDtypeStruct((N_NODES, 2 * EMB), _f32)],
        "embed_tables",
    )(x, wemb, bemb, wd, bd, ws)


def _node_update(node_prev, outlin, sums, wd, bd, ws, make_tables):
    def body(np_r, ol_r, sums_r, wd_r, bd_r, ws_r, node_r, td_r, ts_r):
        s = sums_r[...]
        m = s[0:1, :] / N_NODES
        v = s[1:2, :] / N_NODES - m * m
        node = jax.nn.softplus(np_r[...] + (ol_r[...] - m) / jnp.sqrt(v + 1e-5))
        node_r[...] = node
        td_r[...] = jnp.dot(node, wd_r[...], preferred_element_type=_f32) + bd_r[...]
        ts_r[...] = jnp.dot(node, ws_r[...], preferred_element_type=_f32)

    def body_plain(np_r, ol_r, sums_r, node_r):
        s = sums_r[...]
        m = s[0:1, :] / N_NODES
        v = s[1:2, :] / N_NODES - m * m
        node_r[...] = jax.nn.softplus(
            np_r[...] + (ol_r[...] - m) / jnp.sqrt(v + 1e-5))

    if make_tables:
        return _tc(
            body, (N_NODES // RN,),
            [_rows(RN, EMB), _rows(RN, EMB), _full((2, EMB)),
             _full((EMB, 3 * EMB)), _full((1, 3 * EMB)), _full((EMB, 2 * EMB))],
            [_rows(RN, EMB), _rows(RN, 3 * EMB), _rows(RN, 2 * EMB)],
            [jax.ShapeDtypeStruct((N_NODES, EMB), _f32),
             jax.ShapeDtypeStruct((N_NODES, 3 * EMB), _f32),
             jax.ShapeDtypeStruct((N_NODES, 2 * EMB), _f32)],
            "node_update_tables",
        )(node_prev, outlin, sums, wd, bd, ws)
    return _tc(
        body_plain, (N_NODES // RN,),
        [_rows(RN, EMB), _rows(RN, EMB), _full((2, EMB))],
        [_rows(RN, EMB)],
        [jax.ShapeDtypeStruct((N_NODES, EMB), _f32)],
        "node_update",
    )(node_prev, outlin, sums)[0]


def _quad_weights(vmin, vmax):
    gamma = (BINS - 1) / (vmax - vmin)
    c = jnp.linspace(vmin, vmax, BINS, dtype=_f32)
    rows = jnp.stack([-gamma * jnp.ones((BINS,), _f32),
                      2.0 * gamma * c, -gamma * c * c], axis=0)
    return jnp.concatenate([rows, jnp.zeros((5, BINS), _f32)], axis=0)


def _rbf_expand(d, qw, w, b):
    lane = lax.broadcasted_iota(jnp.int32, (RE, 8), 1)
    a = jnp.where(lane == 0, d * d,
                  jnp.where(lane == 1, d,
                            (lane == 2).astype(_f32)))
    q = lax.dot_general(a, qw, (((1,), (0,)), ((), ())),
                        preferred_element_type=_f32,
                        precision=lax.Precision.HIGHEST)
    ex = jnp.exp(q)
    return jax.nn.softplus(
        jnp.dot(ex, w, preferred_element_type=_f32) + b)


def _rbf_edge(vecs, qw, w, b):
    L = vecs.shape[0]

    def body(v_r, qw_r, w_r, b_r, o_r):
        v = v_r[...]
        d = -0.75 / jnp.sqrt(jnp.sum(v * v, axis=1, keepdims=True))
        o_r[...] = _rbf_expand(d, qw_r[...], w_r[...], b_r[...])

    return _tc(
        body, (L // RE,),
        [_rows(RE, 3), _full((8, BINS)), _full((BINS, EMB)),
         _full((1, EMB))],
        [_rows(RE, EMB)],
        [jax.ShapeDtypeStruct((L, EMB), _f32)],
        "rbf_norm",
    )(vecs, qw, w, b)[0]


def _rbf_nei(nei9, att, qw_len, w_len, b_len, qw_ang, w_ang, b_ang):
    nsteps = N_EDGES // RE

    def body(v_r, a_r, qwl_r, wl_r, bl_r, qwa_r, wa_r, ba_r,
             l0, l1, l2, c0, c1, c2):
        v = v_r[...]
        a = a_r[...]
        vv = v * v
        av = v * jnp.concatenate([a, a, a], axis=1)
        ana = jnp.sqrt(jnp.sum(a * a, axis=1, keepdims=True))
        lane = lax.broadcasted_iota(jnp.int32, (RE, 9), 1)
        louts = (l0, l1, l2)
        couts = (c0, c1, c2)
        for t in range(3):
            m = (lane >= 3 * t) & (lane < 3 * t + 3)
            ss = jnp.sum(jnp.where(m, vv, 0.0), axis=1, keepdims=True)
            nn = jnp.sqrt(ss)
            d = -0.75 / nn
            louts[t][...] = _rbf_expand(d, qwl_r[...], wl_r[...], bl_r[...])
            dot = jnp.sum(jnp.where(m, av, 0.0), axis=1, keepdims=True)
            cos = jnp.clip(dot / (nn * ana), -1.0, 1.0)
            couts[t][...] = _rbf_expand(cos, qwa_r[...], wa_r[...], ba_r[...])

    e_shape = jax.ShapeDtypeStruct((N_EDGES, EMB), _f32)
    return _tc(
        body, (nsteps,),
        [_rows(RE, 9), _rows(RE, 3), _full((8, BINS)), _full((BINS, EMB)),
         _full((1, EMB)), _full((8, BINS)), _full((BINS, EMB)),
         _full((1, EMB))],
        [_rows(RE, EMB)] * 6,
        [e_shape] * 6,
        "rbf_nei",
    )(nei9, att, qw_len, w_len, b_len, qw_ang, w_ang, b_ang)


def _conv_phase1(gd, gs, ef_args, wek, k1, wem, m1, cvec, pre_bn):
    nsteps = N_EDGES // RE

    def compute(gd_r, gs_r, ef, wek_r, k1_r, wem_r, m1_r, cvec_r,
                alpha_r, msg_r, sums_r):
        c = cvec_r[...]
        g = gd_r[...]
        h = gs_r[...]
        qd = g[:, 0:EMB]
        kad = g[:, EMB:2 * EMB]
        vad = g[:, 2 * EMB:3 * EMB]
        kas = h[:, 0:EMB]
        vas = h[:, EMB:2 * EMB]
        hk = jax.nn.silu(kad + kas +
                         jnp.dot(ef, wek_r[...], preferred_element_type=_f32)
                         + c[0:1, :])
        key_j = jnp.dot(hk, k1_r[...], preferred_element_type=_f32) + c[1:2, :]
        alpha = qd * key_j * ISQ
        hm = jax.nn.silu(vad + vas +
                         jnp.dot(ef, wem_r[...], preferred_element_type=_f32)
                         + c[2:3, :])
        msg = jnp.dot(hm, m1_r[...], preferred_element_type=_f32) + c[3:4, :]
        alpha_r[...] = alpha
        msg_r[...] = msg

        @pl.when(pl.program_id(0) == 0)
        def _():
            sums_r[...] = jnp.zeros_like(sums_r)

        part = jnp.concatenate(
            [jnp.sum(alpha, axis=0, keepdims=True),
             jnp.sum(alpha * alpha, axis=0, keepdims=True)], axis=0)
        sums_r[...] = sums_r[...] + part

    out_specs = [_rows(RE, EMB), _rows(RE, EMB), _full((2, EMB))]
    out_shape = [jax.ShapeDtypeStruct((N_EDGES, EMB), _f32),
                 jax.ShapeDtypeStruct((N_EDGES, EMB), _f32),
                 jax.ShapeDtypeStruct((2, EMB), _f32)]

    if not pre_bn:
        (ef,) = ef_args

        def body(gd_r, gs_r, ef_r, wek_r, k1_r, wem_r, m1_r, cvec_r,
                 alpha_r, msg_r, sums_r):
            compute(gd_r, gs_r, ef_r[...], wek_r, k1_r, wem_r, m1_r, cvec_r,
                    alpha_r, msg_r, sums_r)

        return _tc(
            body, (nsteps,),
            [_rows(RE, 3 * EMB), _rows(RE, 2 * EMB), _rows(RE, EMB),
             _full((EMB, EMB)), _full((EMB, EMB)), _full((EMB, EMB)),
             _full((EMB, EMB)), _full((4, EMB))],
            out_specs, out_shape, "conv_phase1",
        )(gd, gs, ef, wek, k1, wem, m1, cvec)

    ef0, outlin_e, esums = ef_args

    def body2(gd_r, gs_r, ef0_r, ol_r, es_r, wek_r, k1_r, wem_r, m1_r,
              cvec_r, alpha_r, msg_r, sums_r):
        s = es_r[...]
        m = s[0:1, :] / N_EDGES
        v = s[1:2, :] / N_EDGES - m * m
        ef = jax.nn.softplus(ef0_r[...] + (ol_r[...] - m) / jnp.sqrt(v + 1e-5))
        compute(gd_r, gs_r, ef, wek_r, k1_r, wem_r, m1_r, cvec_r,
                alpha_r, msg_r, sums_r)

    return _tc(
        body2, (nsteps,),
        [_rows(RE, 3 * EMB), _rows(RE, 2 * EMB), _rows(RE, EMB),
         _rows(RE, EMB), _full((2, EMB)),
         _full((EMB, EMB)), _full((EMB, EMB)), _full((EMB, EMB)),
         _full((EMB, EMB)), _full((4, EMB))],
        out_specs, out_shape, "conv_phase1_bn",
    )(gd, gs, ef0, outlin_e, esums, wek, k1, wem, m1, cvec)


def _conv_gate(alpha, msg, sums):
    def body(a_r, m_r, s_r, o_r):
        s = s_r[...]
        mean = s[0:1, :] / N_EDGES
        var = s[1:2, :] / N_EDGES - mean * mean
        bn = (a_r[...] - mean) / jnp.sqrt(var + 1e-5)
        o_r[...] = m_r[...] * jax.nn.sigmoid(bn)

    return _tc(
        body, (N_EDGES // RE,),
        [_rows(RE, EMB), _rows(RE, EMB), _full((2, EMB))],
        [_rows(RE, EMB)],
        [jax.ShapeDtypeStruct((N_EDGES, EMB), _f32)],
        "conv_gate",
    )(alpha, msg, sums)[0]


def _conv_concate(agg2, wc, bc):
    nsteps = N_NODES // RN

    def body(a_r, b_r, wc_r, bc_r, o_r, sums_r):
        s = a_r[...] + b_r[...]
        o = jnp.dot(s, wc_r[...], preferred_element_type=_f32) + bc_r[...]
        o_r[...] = o

        @pl.when(pl.program_id(0) == 0)
        def _():
            sums_r[...] = jnp.zeros_like(sums_r)

        part = jnp.concatenate(
            [jnp.sum(o, axis=0, keepdims=True),
             jnp.sum(o * o, axis=0, keepdims=True)], axis=0)
        sums_r[...] = sums_r[...] + part

    spec_a = pl.BlockSpec((RN, EMB), lambda i: (i, 0))
    spec_b = pl.BlockSpec((RN, EMB), lambda i: (i + N_NODES // RN, 0))
    return _tc(
        body, (nsteps,),
        [spec_a, spec_b, _full((EMB, EMB)), _full((1, EMB))],
        [_rows(RN, EMB), _full((2, EMB))],
        [jax.ShapeDtypeStruct((N_NODES, EMB), _f32),
         jax.ShapeDtypeStruct((2, EMB), _f32)],
        "conv_concate",
    )(agg2, agg2, wc, bc)


def _edge_phase_a(ef, nls, nas, wq, akx, av, bks, bvs, ck, cv, k1, m1,
                  cks, cvs, oth):
    nsteps = N_EDGES // RE

    def body(ef_r, nl0, nl1, nl2, na0, na1, na2, wq_r, akx_r, av_r,
             bks_r, bvs_r, ck_r, cv_r, k1_r, m1_r, cks_r, cvs_r, oth_r,
             a0, a1, a2, v0, v1, v2, sums_r):
        e = ef_r[...]
        othv = oth_r[...]
        q = jnp.dot(e, wq_r[...], preferred_element_type=_f32) + othv[0:1, :]
        ekx = jnp.dot(e, akx_r[...], preferred_element_type=_f32)
        evx = jnp.dot(e, av_r[...], preferred_element_type=_f32)
        k1v = k1_r[...]
        m1v = m1_r[...]
        ckv = ck_r[...]
        cvv = cv_r[...]
        bksv = bks_r[...]
        bvsv = bvs_r[...]
        cksv = cks_r[...]
        cvsv = cvs_r[...]
        nls = (nl0[...], nl1[...], nl2[...])
        nas = (na0[...], na1[...], na2[...])
        aouts = (a0, a1, a2)
        vouts = (v0, v1, v2)
        ssum = jnp.zeros((1, EMB), _f32)
        ssq = jnp.zeros((1, EMB), _f32)
        for t in range(3):
            bk_t = bksv[t * EMB:(t + 1) * EMB, :]
            bv_t = bvsv[t * EMB:(t + 1) * EMB, :]
            hk = jax.nn.silu(
                ekx + jnp.dot(nls[t], bk_t, preferred_element_type=_f32)
                + jnp.dot(nas[t], ckv, preferred_element_type=_f32)
                + cksv[t:t + 1, :])
            kt = jnp.dot(hk, k1v, preferred_element_type=_f32) + othv[1:2, :]
            at = q * kt * ISQ
            aouts[t][...] = at
            hv = jax.nn.silu(
                evx + jnp.dot(nls[t], bv_t, preferred_element_type=_f32)
                + jnp.dot(nas[t], cvv, preferred_element_type=_f32)
                + cvsv[t:t + 1, :])
            vt = jnp.dot(hv, m1v, preferred_element_type=_f32) + othv[2:3, :]
            vouts[t][...] = vt
            ssum = ssum + jnp.sum(at, axis=0, keepdims=True)
            ssq = ssq + jnp.sum(at * at, axis=0, keepdims=True)

        @pl.when(pl.program_id(0) == 0)
        def _():
            sums_r[...] = jnp.zeros_like(sums_r)

        sums_r[...] = sums_r[...] + jnp.concatenate([ssum, ssq], axis=0)

    e_shape = jax.ShapeDtypeStruct((N_EDGES, EMB), _f32)
    return _tc(
        body, (nsteps,),
        [_rows(RE, EMB)] * 7 +
        [_full((EMB, EMB)), _full((EMB, EMB)), _full((EMB, EMB)),
         _full((3 * EMB, EMB)), _full((3 * EMB, EMB)),
         _full((EMB, EMB)), _full((EMB, EMB)),
         _full((EMB, EMB)), _full((EMB, EMB)),
         _full((3, EMB)), _full((3, EMB)), _full((3, EMB))],
        [_rows(RE, EMB)] * 6 + [_full((2, EMB))],
        [e_shape] * 6 + [jax.ShapeDtypeStruct((2, EMB), _f32)],
        "edge_phase_a",
    )(ef, *nls, *nas, wq, akx, av, bks, bvs, ck, cv, k1, m1,
      cks, cvs, oth)


def _edge_phase_b(alphas, vals, asums, wc, bc3):
    nsteps = N_EDGES // RE

    def body(a0, a1, a2, v0, v1, v2, as_r, wc_r, bc3_r, o_r, sums_r):
        s = as_r[...]
        mean = s[0:1, :] / (3 * N_EDGES)
        var = s[1:2, :] / (3 * N_EDGES) - mean * mean
        rstd = 1.0 / jnp.sqrt(var + 1e-5)
        acc = jnp.zeros((RE, EMB), _f32)
        for a_r, v_r in ((a0, v0), (a1, v1), (a2, v2)):
            gate = jax.nn.sigmoid((a_r[...] - mean) * rstd)
            acc = acc + v_r[...] * gate
        o = jnp.dot(acc, wc_r[...], preferred_element_type=_f32) + bc3_r[...]
        o_r[...] = o

        @pl.when(pl.program_id(0) == 0)
        def _():
            sums_r[...] = jnp.zeros_like(sums_r)

        part = jnp.concatenate(
            [jnp.sum(o, axis=0, keepdims=True),
             jnp.sum(o * o, axis=0, keepdims=True)], axis=0)
        sums_r[...] = sums_r[...] + part

    return _tc(
        body, (nsteps,),
        [_rows(RE, EMB)] * 6 + [_full((2, EMB)), _full((EMB, EMB)),
                                _full((1, EMB))],
        [_rows(RE, EMB), _full((2, EMB))],
        [jax.ShapeDtypeStruct((N_EDGES, EMB), _f32),
         jax.ShapeDtypeStruct((2, EMB), _f32)],
        "edge_phase_b",
    )(*alphas, *vals, asums, wc, bc3)


def _pool_head(node, batch3, w0, b0, w1, b1, w2, b2):
    nsteps = N_NODES // RN

    def body(n_r, bt_r, w0_r, b0_r, w1_r, b1_r, w2_r, b2_r, o_r,
             sacc, cacc):
        @pl.when(pl.program_id(0) == 0)
        def _():
            sacc[...] = jnp.zeros_like(sacc)
            cacc[...] = jnp.zeros_like(cacc)

        b = bt_r[0]
        oh = (lax.broadcasted_iota(jnp.int32, (N_GRAPHS, RN), 0)
              == b).astype(_f32)
        sacc[...] = sacc[...] + lax.dot_general(
            oh, n_r[...], (((1,), (0,)), ((), ())),
            preferred_element_type=_f32)
        cacc[...] = cacc[...] + jnp.broadcast_to(
            jnp.sum(oh, axis=1, keepdims=True), (N_GRAPHS, EMB))

        @pl.when(pl.program_id(0) == nsteps - 1)
        def _():
            feats = sacc[...] / jnp.maximum(cacc[...], 1.0)
            h = jax.nn.silu(
                jnp.dot(feats, w0_r[...], preferred_element_type=_f32)
                + b0_r[...])
            h = jax.nn.silu(
                jnp.dot(h, w1_r[...], preferred_element_type=_f32)
                + b1_r[...])
            o_r[...] = (jnp.dot(h, w2_r[...], preferred_element_type=_f32)
                        + b2_r[...])

    return pl.pallas_call(
        body,
        grid=(nsteps,),
        in_specs=[_rows(RN, EMB),
                  pl.BlockSpec((1, 1, RN), lambda i: (i, 0, 0)),
                  _full((EMB, EMB)), _full((1, EMB)),
                  _full((EMB, EMB)), _full((1, EMB)),
                  _full((EMB, 6)), _full((1, 6))],
        out_specs=_full((N_GRAPHS, 6)),
        out_shape=jax.ShapeDtypeStruct((N_GRAPHS, 6), _f32),
        scratch_shapes=[pltpu.VMEM((N_GRAPHS, EMB), _f32),
                        pltpu.VMEM((N_GRAPHS, EMB), _f32)],
        compiler_params=pltpu.CompilerParams(
            dimension_semantics=("arbitrary",)),
        name="pool_head",
    )(node, batch3, w0, b0, w1, b1, w2, b2)


def _sc_gather(table, idx_pad, width):
    mesh = plsc.VectorSubcoreMesh(core_axis_name="c", subcore_axis_name="s")

    @functools.partial(
        pl.kernel,
        out_type=jax.ShapeDtypeStruct((N_EDGES, width), _f32),
        mesh=mesh,
        scratch_types=[pltpu.VMEM((CPW * CHUNK,), jnp.int32),
                       pltpu.VMEM((CHUNK, width), _f32),
                       pltpu.VMEM((CHUNK, width), _f32),
                       pltpu.SemaphoreType.DMA,
                       pltpu.SemaphoreType.DMA],
    )
    def k(table_hbm, idx_hbm, out_hbm, idx_v, buf0, buf1, s0, s1):
        w = lax.axis_index("s") * NC + lax.axis_index("c")
        base = w * (CPW * CHUNK)
        pltpu.sync_copy(idx_hbm.at[pl.ds(base, CPW * CHUNK)], idx_v)
        nch = jnp.minimum(CPW, TOTAL_CHUNKS - w * CPW)
        nch2 = nch // 2

        def gather(j, buf, sem):
            return pltpu.async_copy(
                table_hbm.at[idx_v.at[pl.ds(j * CHUNK, CHUNK)]], buf, sem)

        gather(0, buf0, s0)

        def body(g, carry):
            j0 = 2 * g
            pltpu.make_async_copy(
                table_hbm.at[idx_v.at[pl.ds(j0 * CHUNK, CHUNK)]],
                buf0, s0).wait()
            gather(j0 + 1, buf1, s1)
            pltpu.sync_copy(buf0,
                            out_hbm.at[pl.ds(base + j0 * CHUNK, CHUNK)])
            pltpu.make_async_copy(
                table_hbm.at[idx_v.at[pl.ds((j0 + 1) * CHUNK, CHUNK)]],
                buf1, s1).wait()

            @pl.when(g + 1 < nch2)
            def _():
                gather(j0 + 2, buf0, s0)

            pltpu.sync_copy(
                buf1, out_hbm.at[pl.ds(base + (j0 + 1) * CHUNK, CHUNK)])
            return carry

        lax.fori_loop(0, nch2, body, 0)

    return k(table, idx_pad)


def _sc_scatter_add(gated, idx3, zrows):
    mesh = plsc.VectorSubcoreMesh(core_axis_name="c", subcore_axis_name="s")

    NPAD = 10240

    @functools.partial(
        pl.kernel,
        out_type=jax.ShapeDtypeStruct((2 * N_NODES, EMB), _f32),
        mesh=mesh,
        scratch_types=[pltpu.VMEM_SHARED((NPAD, EMB), _f32),
                       pltpu.VMEM((CPW, CHUNK), jnp.int32),
                       pltpu.VMEM((CHUNK, EMB), _f32),
                       pltpu.VMEM((CHUNK, EMB), _f32),
                       pltpu.SemaphoreType.DMA,
                       pltpu.SemaphoreType.DMA],
    )
    def k(g_hbm, idx_hbm, z_hbm, out_hbm, acc, idx_v, buf0, buf1, s0, s1):
        cid = lax.axis_index("c")
        sid = lax.axis_index("s")
        w = sid * NC + cid
        row0 = sid * 640
        pltpu.sync_copy(z_hbm, acc.at[pl.ds(row0, 640)])
        plsc.subcore_barrier()
        pltpu.sync_copy(idx_hbm.at[w], idx_v)
        base = w * (CPW * CHUNK)
        nch = jnp.minimum(CPW, TOTAL_CHUNKS - w * CPW)
        nch2 = nch // 2

        def load(j, buf, sem):
            return pltpu.async_copy(
                g_hbm.at[pl.ds(base + j * CHUNK, CHUNK)], buf, sem)

        load(0, buf0, s0)

        def body(g, carry):
            j0 = 2 * g
            pltpu.make_async_copy(
                g_hbm.at[pl.ds(base + j0 * CHUNK, CHUNK)], buf0, s0).wait()
            load(j0 + 1, buf1, s1)
            pltpu.sync_copy(buf0, acc.at[idx_v.at[j0]], add=True)
            pltpu.make_async_copy(
                g_hbm.at[pl.ds(base + (j0 + 1) * CHUNK, CHUNK)],
                buf1, s1).wait()

            @pl.when(g + 1 < nch2)
            def _():
                load(j0 + 2, buf0, s0)

            pltpu.sync_copy(buf1, acc.at[idx_v.at[j0 + 1]], add=True)
            return carry

        lax.fori_loop(0, nch2, body, 0)
        plsc.subcore_barrier()

        @pl.when(sid < NS - 1)
        def _():
            pltpu.sync_copy(acc.at[pl.ds(row0, 640)],
                            out_hbm.at[pl.ds(cid * N_NODES + row0, 640)])

        @pl.when(sid == NS - 1)
        def _():
            pltpu.sync_copy(acc.at[pl.ds(row0, 400)],
                            out_hbm.at[pl.ds(cid * N_NODES + row0, 400)])

    return k(gated, idx3, zrows)


def _compose_conv(cp):
    wq, bq = cp["lin_query"]["w"], cp["lin_query"]["b"]
    wk, bk = cp["lin_key"]["w"], cp["lin_key"]["b"]
    wv, bv = cp["lin_value"]["w"], cp["lin_value"]["b"]
    we, be = cp["lin_edge"]["w"], cp["lin_edge"]["b"]
    k0, k0b = cp["key_update"]["l0"]["w"], cp["key_update"]["l0"]["b"]
    k1, k1b = cp["key_update"]["l1"]["w"], cp["key_update"]["l1"]["b"]
    m0, m0b = cp["lin_msg_update"]["l0"]["w"], cp["lin_msg_update"]["l0"]["b"]
    m1, m1b = cp["lin_msg_update"]["l1"]["w"], cp["lin_msg_update"]["l1"]["b"]
    k0i, k0j, k0e = k0[:EMB], k0[EMB:2 * EMB], k0[2 * EMB:]
    m0i, m0j, m0e = m0[:EMB], m0[EMB:2 * EMB], m0[2 * EMB:]
    wd = jnp.concatenate([wq, wk @ k0i, wv @ m0i], axis=1)
    bd = jnp.concatenate(
        [bq, jnp.zeros((EMB,), _f32), jnp.zeros((EMB,), _f32)])[None, :]
    ws = jnp.concatenate([wk @ k0j, wv @ m0j], axis=1)
    wek = we @ k0e
    wem = we @ m0e
    ck = k0b + bk @ k0i + bk @ k0j + be @ k0e
    cm = m0b + bv @ m0i + bv @ m0j + be @ m0e
    cvec = jnp.stack([ck, k1b, cm, m1b], axis=0)
    return dict(wd=wd, bd=bd, ws=ws, wek=wek, wem=wem, k1=k1, m1=m1,
                cvec=cvec, wc=cp["lin_concate"]["w"],
                bc=cp["lin_concate"]["b"][None, :])


def _compose_edge(cp):
    wq, bq = cp["lin_query"]["w"], cp["lin_query"]["b"]
    wk, bk = cp["lin_key"]["w"], cp["lin_key"]["b"]
    wv, bv = cp["lin_value"]["w"], cp["lin_value"]["b"]
    we = cp["lin_edge"]["w"]
    k0, k0b = cp["key_update"]["l0"]["w"], cp["key_update"]["l0"]["b"]
    k1, k1b = cp["key_update"]["l1"]["w"], cp["key_update"]["l1"]["b"]
    m0, m0b = cp["lin_msg_update"]["l0"]["w"], cp["lin_msg_update"]["l0"]["b"]
    m1, m1b = cp["lin_msg_update"]["l1"]["w"], cp["lin_msg_update"]["l1"]["b"]
    k0x, k0y, k0e = k0[:EMB], k0[EMB:2 * EMB], k0[2 * EMB:]
    m0x, m0y, m0e = m0[:EMB], m0[EMB:2 * EMB], m0[2 * EMB:]
    akx = wk @ k0x
    av = wv @ m0x
    ckm = we @ k0e
    cvm = we @ m0e
    bks, bvs, cks, cvs = [], [], [], []
    for t, (ke, ve) in enumerate((("lin_key_e1", "lin_value_e1"),
                                  ("lin_key_e2", "lin_value_e2"),
                                  ("lin_key_e3", "lin_value_e3"))):
        wke, bke = cp[ke]["w"], cp[ke]["b"]
        wve, bve = cp[ve]["w"], cp[ve]["b"]
        bks.append(wke @ k0y)
        bvs.append(wve @ m0y)
        cks.append(k0b + bk @ k0x + bke @ k0y)
        cvs.append(m0b + bv @ m0x + bve @ m0y)
    oth = jnp.stack([bq, k1b, m1b], axis=0)
    return dict(wq=wq, akx=akx, av=av,
                bks=jnp.concatenate(bks, axis=0),
                bvs=jnp.concatenate(bvs, axis=0),
                ck=ckm, cv=cvm, k1=k1, m1=m1,
                cks=jnp.stack(cks, axis=0), cvs=jnp.stack(cvs, axis=0),
                oth=oth, wc=cp["lin_concate"]["w"],
                bc3=3.0 * cp["lin_concate"]["b"][None, :])


def _conv_layer(comp, td, ts, ef_args, src_pad, dst_pad, dst3, zrows,
                node_prev, next_comp):
    gd = _sc_gather(td, dst_pad, 3 * EMB)
    gs = _sc_gather(ts, src_pad, 2 * EMB)
    alpha, msg, asums = _conv_phase1(
        gd, gs, ef_args, comp["wek"], comp["k1"], comp["wem"], comp["m1"],
        comp["cvec"], pre_bn=(len(ef_args) == 3))
    gated = _conv_gate(alpha, msg, asums)
    agg2 = _sc_scatter_add(gated, dst3, zrows)
    outlin, osums = _conv_concate(agg2, comp["wc"], comp["bc"])
    if next_comp is not None:
        return _node_update(node_prev, outlin, osums, next_comp["wd"],
                            next_comp["bd"], next_comp["ws"],
                            make_tables=True)
    return _node_update(node_prev, outlin, osums, None, None, None,
                        make_tables=False)


def kernel(x, edge_index, edge_attr, edge_nei, batch, params):
    src = edge_index[0]
    dst = edge_index[1]
    comp0 = _compose_conv(params["att0"])
    comp1 = _compose_conv(params["att1"])
    compe = _compose_edge(params["edge_update"])
    wrbf, brbf = params["rbf"]["w"], params["rbf"]["b"][None, :]
    wrba, brba = params["rbf_angle"]["w"], params["rbf_angle"]["b"][None, :]

    pad = EPAD - N_EDGES
    src_pad = jnp.pad(src, (0, pad))
    dst_pad = jnp.pad(dst, (0, pad))
    dst3 = dst_pad.reshape(NW, CPW, CHUNK)
    zrows = jnp.zeros((640, EMB), _f32)
    batch3 = batch.reshape(N_NODES // RN, 1, RN)
    qw_len = _quad_weights(-4.0, 0.0)
    qw_ang = _quad_weights(-1.0, 1.0)

    ef0 = _rbf_edge(edge_attr, qw_len, wrbf, brbf)
    nl0, nl1, nl2, na0, na1, na2 = _rbf_nei(
        edge_nei.reshape(N_EDGES, 9), edge_attr,
        qw_len, wrbf, brbf, qw_ang, wrba, brba)

    node0, td0, ts0 = _embed_tables(
        x, params["atom_embedding"]["w"],
        params["atom_embedding"]["b"][None, :],
        comp0["wd"], comp0["bd"], comp0["ws"])

    node1, td1, ts1 = _conv_layer(
        comp0, td0, ts0, (ef0,), src_pad, dst_pad, dst3, zrows,
        node0, comp1)

    alphas_vals = _edge_phase_a(
        ef0, (nl0, nl1, nl2), (na0, na1, na2),
        compe["wq"], compe["akx"], compe["av"], compe["bks"],
        compe["bvs"], compe["ck"], compe["cv"], compe["k1"], compe["m1"],
        compe["cks"], compe["cvs"], compe["oth"])
    a0, a1, a2, v0, v1, v2, easums = alphas_vals
    outlin_e, esums = _edge_phase_b((a0, a1, a2), (v0, v1, v2), easums,
                                    compe["wc"], compe["bc3"])

    node2 = _conv_layer(
        comp1, td1, ts1, (ef0, outlin_e, esums), src_pad, dst_pad, dst3,
        zrows, node1, None)

    out = _pool_head(node2, batch3,
                     params["fc0"]["w"], params["fc0"]["b"][None, :],
                     params["fc1"]["w"], params["fc1"]["b"][None, :],
                     params["fc_out"]["w"], params["fc_out"]["b"][None, :])
    return jnp.squeeze(out)


# speedup vs baseline: 2.1111x; 1.2326x over previous
import functools
import math

import jax
import jax.numpy as jnp
from jax import lax
from jax.experimental import pallas as pl
from jax.experimental.pallas import tpu as pltpu
from jax.experimental.pallas import tpu_sc as plsc

N_NODES = 10000
N_EDGES = 160000
N_GRAPHS = 64
EMB = 128
BINS = 512

RE = 640
RN = 1000
NC, NS = 2, 16
NW = NC * NS
CHUNK = 128
CPW = 40
TOTAL_CHUNKS = N_EDGES // CHUNK
EPAD = NW * CPW * CHUNK
ROWS_PER_TILE = N_NODES // NS
ISQ = 1.0 / math.sqrt(EMB)

_f32 = jnp.float32


def _tc(body, grid, in_specs, out_specs, out_shape, name):
    return pl.pallas_call(
        body,
        grid=grid,
        in_specs=in_specs,
        out_specs=out_specs,
        out_shape=out_shape,
        compiler_params=pltpu.CompilerParams(
            dimension_semantics=("arbitrary",)),
        name=name,
    )


def _full(shape):
    return pl.BlockSpec(shape, lambda i: (0,) * len(shape))


def _rows(r, cols):
    return pl.BlockSpec((r, cols), lambda i: (i, 0))


def _embed_tables(x, wemb, bemb, wd, bd, ws):
    def body(x_r, wemb_r, bemb_r, wd_r, bd_r, ws_r, node_r, td_r, ts_r):
        node = jnp.dot(x_r[...], wemb_r[...], preferred_element_type=_f32)
        node = node + bemb_r[...]
        node_r[...] = node
        td_r[...] = jnp.dot(node, wd_r[...], preferred_element_type=_f32) + bd_r[...]
        ts_r[...] = jnp.dot(node, ws_r[...], preferred_element_type=_f32)

    return _tc(
        body, (N_NODES // RN,),
        [_rows(RN, 92), _full((92, EMB)), _full((1, EMB)),
         _full((EMB, 3 * EMB)), _full((1, 3 * EMB)), _full((EMB, 2 * EMB))],
        [_rows(RN, EMB), _rows(RN, 3 * EMB), _rows(RN, 2 * EMB)],
        [jax.ShapeDtypeStruct((N_NODES, EMB), _f32),
         jax.ShapeDtypeStruct((N_NODES, 3 * EMB), _f32),
         jax.ShapeDtypeStruct((N_NODES, 2 * EMB), _f32)],
        "embed_tables",
    )(x, wemb, bemb, wd, bd, ws)


def _node_update(node_prev, outlin, sums, wd, bd, ws, make_tables):
    def body(np_r, ol_r, sums_r, wd_r, bd_r, ws_r, node_r, td_r, ts_r):
        s = sums_r[...]
        m = s[0:1, :] / N_NODES
        v = s[1:2, :] / N_NODES - m * m
        node = jax.nn.softplus(np_r[...] + (ol_r[...] - m) / jnp.sqrt(v + 1e-5))
        node_r[...] = node
        td_r[...] = jnp.dot(node, wd_r[...], preferred_element_type=_f32) + bd_r[...]
        ts_r[...] = jnp.dot(node, ws_r[...], preferred_element_type=_f32)

    def body_plain(np_r, ol_r, sums_r, node_r):
        s = sums_r[...]
        m = s[0:1, :] / N_NODES
        v = s[1:2, :] / N_NODES - m * m
        node_r[...] = jax.nn.softplus(
            np_r[...] + (ol_r[...] - m) / jnp.sqrt(v + 1e-5))

    if make_tables:
        return _tc(
            body, (N_NODES // RN,),
            [_rows(RN, EMB), _rows(RN, EMB), _full((2, EMB)),
             _full((EMB, 3 * EMB)), _full((1, 3 * EMB)), _full((EMB, 2 * EMB))],
            [_rows(RN, EMB), _rows(RN, 3 * EMB), _rows(RN, 2 * EMB)],
            [jax.ShapeDtypeStruct((N_NODES, EMB), _f32),
             jax.ShapeDtypeStruct((N_NODES, 3 * EMB), _f32),
             jax.ShapeDtypeStruct((N_NODES, 2 * EMB), _f32)],
            "node_update_tables",
        )(node_prev, outlin, sums, wd, bd, ws)
    return _tc(
        body_plain, (N_NODES // RN,),
        [_rows(RN, EMB), _rows(RN, EMB), _full((2, EMB))],
        [_rows(RN, EMB)],
        [jax.ShapeDtypeStruct((N_NODES, EMB), _f32)],
        "node_update",
    )(node_prev, outlin, sums)[0]


def _rbf_expand(d, vmin, vmax, w, b):
    gamma = (BINS - 1) / (vmax - vmin)
    step = (vmax - vmin) / (BINS - 1)
    kk = -gamma * 1.4426950408889634
    c = vmin + step * lax.broadcasted_iota(
        jnp.int32, (RE, BINS), 1).astype(_f32)
    u = d - c
    ex = jnp.exp2((kk * u) * u)
    return jax.nn.softplus(
        jnp.dot(ex, w, preferred_element_type=_f32) + b)


def _rbf_edge(vecs, w, b):
    L = vecs.shape[0]

    def body(v_r, w_r, b_r, o_r):
        v = v_r[...]
        d = -0.75 / jnp.sqrt(jnp.sum(v * v, axis=1, keepdims=True))
        o_r[...] = _rbf_expand(d, -4.0, 0.0, w_r[...], b_r[...])

    return _tc(
        body, (L // RE,),
        [_rows(RE, 3), _full((BINS, EMB)), _full((1, EMB))],
        [_rows(RE, EMB)],
        [jax.ShapeDtypeStruct((L, EMB), _f32)],
        "rbf_norm",
    )(vecs, w, b)[0]


def _rbf_nei(nei9, att, w_len, b_len, w_ang, b_ang):
    nsteps = N_EDGES // RE

    def body(v_r, a_r, wl_r, bl_r, wa_r, ba_r,
             l0, l1, l2, c0, c1, c2):
        v = v_r[...]
        a = a_r[...]
        vv = v * v
        av = v * jnp.concatenate([a, a, a], axis=1)
        ana = jnp.sqrt(jnp.sum(a * a, axis=1, keepdims=True))
        lane = lax.broadcasted_iota(jnp.int32, (RE, 9), 1)
        louts = (l0, l1, l2)
        couts = (c0, c1, c2)
        for t in range(3):
            m = (lane >= 3 * t) & (lane < 3 * t + 3)
            ss = jnp.sum(jnp.where(m, vv, 0.0), axis=1, keepdims=True)
            nn = jnp.sqrt(ss)
            d = -0.75 / nn
            louts[t][...] = _rbf_expand(d, -4.0, 0.0, wl_r[...], bl_r[...])
            dot = jnp.sum(jnp.where(m, av, 0.0), axis=1, keepdims=True)
            cos = jnp.clip(dot / (nn * ana), -1.0, 1.0)
            couts[t][...] = _rbf_expand(cos, -1.0, 1.0, wa_r[...], ba_r[...])

    e_shape = jax.ShapeDtypeStruct((N_EDGES, EMB), _f32)
    return _tc(
        body, (nsteps,),
        [_rows(RE, 9), _rows(RE, 3), _full((BINS, EMB)),
         _full((1, EMB)), _full((BINS, EMB)), _full((1, EMB))],
        [_rows(RE, EMB)] * 6,
        [e_shape] * 6,
        "rbf_nei",
    )(nei9, att, w_len, b_len, w_ang, b_ang)


def _conv_phase1(gd, gs, ef_args, wek, k1, wem, m1, cvec, pre_bn):
    nsteps = N_EDGES // RE

    def compute(gd_r, gs_r, ef, wek_r, k1_r, wem_r, m1_r, cvec_r,
                alpha_r, msg_r, sums_r):
        c = cvec_r[...]
        g = gd_r[...]
        h = gs_r[...]
        qd = g[:, 0:EMB]
        kad = g[:, EMB:2 * EMB]
        vad = g[:, 2 * EMB:3 * EMB]
        kas = h[:, 0:EMB]
        vas = h[:, EMB:2 * EMB]
        hk = jax.nn.silu(kad + kas +
                         jnp.dot(ef, wek_r[...], preferred_element_type=_f32)
                         + c[0:1, :])
        key_j = jnp.dot(hk, k1_r[...], preferred_element_type=_f32) + c[1:2, :]
        alpha = qd * key_j * ISQ
        hm = jax.nn.silu(vad + vas +
                         jnp.dot(ef, wem_r[...], preferred_element_type=_f32)
                         + c[2:3, :])
        msg = jnp.dot(hm, m1_r[...], preferred_element_type=_f32) + c[3:4, :]
        alpha_r[...] = alpha
        msg_r[...] = msg

        @pl.when(pl.program_id(0) == 0)
        def _():
            sums_r[...] = jnp.zeros_like(sums_r)

        part = jnp.concatenate(
            [jnp.sum(alpha, axis=0, keepdims=True),
             jnp.sum(alpha * alpha, axis=0, keepdims=True)], axis=0)
        sums_r[...] = sums_r[...] + part

    out_specs = [_rows(RE, EMB), _rows(RE, EMB), _full((2, EMB))]
    out_shape = [jax.ShapeDtypeStruct((N_EDGES, EMB), _f32),
                 jax.ShapeDtypeStruct((N_EDGES, EMB), _f32),
                 jax.ShapeDtypeStruct((2, EMB), _f32)]

    if not pre_bn:
        (ef,) = ef_args

        def body(gd_r, gs_r, ef_r, wek_r, k1_r, wem_r, m1_r, cvec_r,
                 alpha_r, msg_r, sums_r):
            compute(gd_r, gs_r, ef_r[...], wek_r, k1_r, wem_r, m1_r, cvec_r,
                    alpha_r, msg_r, sums_r)

        return _tc(
            body, (nsteps,),
            [_rows(RE, 3 * EMB), _rows(RE, 2 * EMB), _rows(RE, EMB),
             _full((EMB, EMB)), _full((EMB, EMB)), _full((EMB, EMB)),
             _full((EMB, EMB)), _full((4, EMB))],
            out_specs, out_shape, "conv_phase1",
        )(gd, gs, ef, wek, k1, wem, m1, cvec)

    ef0, outlin_e, esums = ef_args

    def body2(gd_r, gs_r, ef0_r, ol_r, es_r, wek_r, k1_r, wem_r, m1_r,
              cvec_r, alpha_r, msg_r, sums_r):
        s = es_r[...]
        m = s[0:1, :] / N_EDGES
        v = s[1:2, :] / N_EDGES - m * m
        ef = jax.nn.softplus(ef0_r[...] + (ol_r[...] - m) / jnp.sqrt(v + 1e-5))
        compute(gd_r, gs_r, ef, wek_r, k1_r, wem_r, m1_r, cvec_r,
                alpha_r, msg_r, sums_r)

    return _tc(
        body2, (nsteps,),
        [_rows(RE, 3 * EMB), _rows(RE, 2 * EMB), _rows(RE, EMB),
         _rows(RE, EMB), _full((2, EMB)),
         _full((EMB, EMB)), _full((EMB, EMB)), _full((EMB, EMB)),
         _full((EMB, EMB)), _full((4, EMB))],
        out_specs, out_shape, "conv_phase1_bn",
    )(gd, gs, ef0, outlin_e, esums, wek, k1, wem, m1, cvec)


def _conv_gate(alpha, msg, sums):
    def body(a_r, m_r, s_r, o_r):
        s = s_r[...]
        mean = s[0:1, :] / N_EDGES
        var = s[1:2, :] / N_EDGES - mean * mean
        bn = (a_r[...] - mean) / jnp.sqrt(var + 1e-5)
        o_r[...] = m_r[...] * jax.nn.sigmoid(bn)

    return _tc(
        body, (N_EDGES // RE,),
        [_rows(RE, EMB), _rows(RE, EMB), _full((2, EMB))],
        [_rows(RE, EMB)],
        [jax.ShapeDtypeStruct((N_EDGES, EMB), _f32)],
        "conv_gate",
    )(alpha, msg, sums)[0]


def _conv_concate(agg2, wc, bc):
    nsteps = N_NODES // RN

    def body(a_r, b_r, wc_r, bc_r, o_r, sums_r):
        s = a_r[...] + b_r[...]
        o = jnp.dot(s, wc_r[...], preferred_element_type=_f32) + bc_r[...]
        o_r[...] = o

        @pl.when(pl.program_id(0) == 0)
        def _():
            sums_r[...] = jnp.zeros_like(sums_r)

        part = jnp.concatenate(
            [jnp.sum(o, axis=0, keepdims=True),
             jnp.sum(o * o, axis=0, keepdims=True)], axis=0)
        sums_r[...] = sums_r[...] + part

    spec_a = pl.BlockSpec((RN, EMB), lambda i: (i, 0))
    spec_b = pl.BlockSpec((RN, EMB), lambda i: (i + N_NODES // RN, 0))
    return _tc(
        body, (nsteps,),
        [spec_a, spec_b, _full((EMB, EMB)), _full((1, EMB))],
        [_rows(RN, EMB), _full((2, EMB))],
        [jax.ShapeDtypeStruct((N_NODES, EMB), _f32),
         jax.ShapeDtypeStruct((2, EMB), _f32)],
        "conv_concate",
    )(agg2, agg2, wc, bc)


def _edge_phase_a(ef, nls, nas, wq, akx, av, bks, bvs, ck, cv, k1, m1,
                  cks, cvs, oth):
    nsteps = N_EDGES // RE

    def body(ef_r, nl0, nl1, nl2, na0, na1, na2, wq_r, akx_r, av_r,
             bks_r, bvs_r, ck_r, cv_r, k1_r, m1_r, cks_r, cvs_r, oth_r,
             a0, a1, a2, v0, v1, v2, sums_r):
        e = ef_r[...]
        othv = oth_r[...]
        q = jnp.dot(e, wq_r[...], preferred_element_type=_f32) + othv[0:1, :]
        ekx = jnp.dot(e, akx_r[...], preferred_element_type=_f32)
        evx = jnp.dot(e, av_r[...], preferred_element_type=_f32)
        k1v = k1_r[...]
        m1v = m1_r[...]
        ckv = ck_r[...]
        cvv = cv_r[...]
        bksv = bks_r[...]
        bvsv = bvs_r[...]
        cksv = cks_r[...]
        cvsv = cvs_r[...]
        nls = (nl0[...], nl1[...], nl2[...])
        nas = (na0[...], na1[...], na2[...])
        aouts = (a0, a1, a2)
        vouts = (v0, v1, v2)
        ssum = jnp.zeros((1, EMB), _f32)
        ssq = jnp.zeros((1, EMB), _f32)
        for t in range(3):
            bk_t = bksv[t * EMB:(t + 1) * EMB, :]
            bv_t = bvsv[t * EMB:(t + 1) * EMB, :]
            hk = jax.nn.silu(
                ekx + jnp.dot(nls[t], bk_t, preferred_element_type=_f32)
                + jnp.dot(nas[t], ckv, preferred_element_type=_f32)
                + cksv[t:t + 1, :])
            kt = jnp.dot(hk, k1v, preferred_element_type=_f32) + othv[1:2, :]
            at = q * kt * ISQ
            aouts[t][...] = at
            hv = jax.nn.silu(
                evx + jnp.dot(nls[t], bv_t, preferred_element_type=_f32)
                + jnp.dot(nas[t], cvv, preferred_element_type=_f32)
                + cvsv[t:t + 1, :])
            vt = jnp.dot(hv, m1v, preferred_element_type=_f32) + othv[2:3, :]
            vouts[t][...] = vt
            ssum = ssum + jnp.sum(at, axis=0, keepdims=True)
            ssq = ssq + jnp.sum(at * at, axis=0, keepdims=True)

        @pl.when(pl.program_id(0) == 0)
        def _():
            sums_r[...] = jnp.zeros_like(sums_r)

        sums_r[...] = sums_r[...] + jnp.concatenate([ssum, ssq], axis=0)

    e_shape = jax.ShapeDtypeStruct((N_EDGES, EMB), _f32)
    return _tc(
        body, (nsteps,),
        [_rows(RE, EMB)] * 7 +
        [_full((EMB, EMB)), _full((EMB, EMB)), _full((EMB, EMB)),
         _full((3 * EMB, EMB)), _full((3 * EMB, EMB)),
         _full((EMB, EMB)), _full((EMB, EMB)),
         _full((EMB, EMB)), _full((EMB, EMB)),
         _full((3, EMB)), _full((3, EMB)), _full((3, EMB))],
        [_rows(RE, EMB)] * 6 + [_full((2, EMB))],
        [e_shape] * 6 + [jax.ShapeDtypeStruct((2, EMB), _f32)],
        "edge_phase_a",
    )(ef, *nls, *nas, wq, akx, av, bks, bvs, ck, cv, k1, m1,
      cks, cvs, oth)


def _edge_phase_b(alphas, vals, asums, wc, bc3):
    nsteps = N_EDGES // RE

    def body(a0, a1, a2, v0, v1, v2, as_r, wc_r, bc3_r, o_r, sums_r):
        s = as_r[...]
        mean = s[0:1, :] / (3 * N_EDGES)
        var = s[1:2, :] / (3 * N_EDGES) - mean * mean
        rstd = 1.0 / jnp.sqrt(var + 1e-5)
        acc = jnp.zeros((RE, EMB), _f32)
        for a_r, v_r in ((a0, v0), (a1, v1), (a2, v2)):
            gate = jax.nn.sigmoid((a_r[...] - mean) * rstd)
            acc = acc + v_r[...] * gate
        o = jnp.dot(acc, wc_r[...], preferred_element_type=_f32) + bc3_r[...]
        o_r[...] = o

        @pl.when(pl.program_id(0) == 0)
        def _():
            sums_r[...] = jnp.zeros_like(sums_r)

        part = jnp.concatenate(
            [jnp.sum(o, axis=0, keepdims=True),
             jnp.sum(o * o, axis=0, keepdims=True)], axis=0)
        sums_r[...] = sums_r[...] + part

    return _tc(
        body, (nsteps,),
        [_rows(RE, EMB)] * 6 + [_full((2, EMB)), _full((EMB, EMB)),
                                _full((1, EMB))],
        [_rows(RE, EMB), _full((2, EMB))],
        [jax.ShapeDtypeStruct((N_EDGES, EMB), _f32),
         jax.ShapeDtypeStruct((2, EMB), _f32)],
        "edge_phase_b",
    )(*alphas, *vals, asums, wc, bc3)


def _pool_head(node, batch3, w0, b0, w1, b1, w2, b2):
    nsteps = N_NODES // RN

    def body(n_r, bt_r, w0_r, b0_r, w1_r, b1_r, w2_r, b2_r, o_r,
             sacc, cacc):
        @pl.when(pl.program_id(0) == 0)
        def _():
            sacc[...] = jnp.zeros_like(sacc)
            cacc[...] = jnp.zeros_like(cacc)

        b = bt_r[0]
        oh = (lax.broadcasted_iota(jnp.int32, (N_GRAPHS, RN), 0)
              == b).astype(_f32)
        sacc[...] = sacc[...] + lax.dot_general(
            oh, n_r[...], (((1,), (0,)), ((), ())),
            preferred_element_type=_f32)
        cacc[...] = cacc[...] + jnp.broadcast_to(
            jnp.sum(oh, axis=1, keepdims=True), (N_GRAPHS, EMB))

        @pl.when(pl.program_id(0) == nsteps - 1)
        def _():
            feats = sacc[...] / jnp.maximum(cacc[...], 1.0)
            h = jax.nn.silu(
                jnp.dot(feats, w0_r[...], preferred_element_type=_f32)
                + b0_r[...])
            h = jax.nn.silu(
                jnp.dot(h, w1_r[...], preferred_element_type=_f32)
                + b1_r[...])
            o_r[...] = (jnp.dot(h, w2_r[...], preferred_element_type=_f32)
                        + b2_r[...])

    return pl.pallas_call(
        body,
        grid=(nsteps,),
        in_specs=[_rows(RN, EMB),
                  pl.BlockSpec((1, 1, RN), lambda i: (i, 0, 0)),
                  _full((EMB, EMB)), _full((1, EMB)),
                  _full((EMB, EMB)), _full((1, EMB)),
                  _full((EMB, 6)), _full((1, 6))],
        out_specs=_full((N_GRAPHS, 6)),
        out_shape=jax.ShapeDtypeStruct((N_GRAPHS, 6), _f32),
        scratch_shapes=[pltpu.VMEM((N_GRAPHS, EMB), _f32),
                        pltpu.VMEM((N_GRAPHS, EMB), _f32)],
        compiler_params=pltpu.CompilerParams(
            dimension_semantics=("arbitrary",)),
        name="pool_head",
    )(node, batch3, w0, b0, w1, b1, w2, b2)


def _sc_gather(table, idx_pad, width):
    mesh = plsc.VectorSubcoreMesh(core_axis_name="c", subcore_axis_name="s")

    @functools.partial(
        pl.kernel,
        out_type=jax.ShapeDtypeStruct((N_EDGES, width), _f32),
        mesh=mesh,
        scratch_types=[pltpu.VMEM((CPW * CHUNK,), jnp.int32),
                       pltpu.VMEM((CHUNK, width), _f32),
                       pltpu.VMEM((CHUNK, width), _f32),
                       pltpu.SemaphoreType.DMA,
                       pltpu.SemaphoreType.DMA],
    )
    def k(table_hbm, idx_hbm, out_hbm, idx_v, buf0, buf1, s0, s1):
        w = lax.axis_index("s") * NC + lax.axis_index("c")
        base = w * (CPW * CHUNK)
        pltpu.sync_copy(idx_hbm.at[pl.ds(base, CPW * CHUNK)], idx_v)
        nch = jnp.minimum(CPW, TOTAL_CHUNKS - w * CPW)
        nch2 = nch // 2

        def gather(j, buf, sem):
            return pltpu.async_copy(
                table_hbm.at[idx_v.at[pl.ds(j * CHUNK, CHUNK)]], buf, sem)

        gather(0, buf0, s0)

        def body(g, carry):
            j0 = 2 * g
            pltpu.make_async_copy(
                table_hbm.at[idx_v.at[pl.ds(j0 * CHUNK, CHUNK)]],
                buf0, s0).wait()
            gather(j0 + 1, buf1, s1)
            pltpu.sync_copy(buf0,
                            out_hbm.at[pl.ds(base + j0 * CHUNK, CHUNK)])
            pltpu.make_async_copy(
                table_hbm.at[idx_v.at[pl.ds((j0 + 1) * CHUNK, CHUNK)]],
                buf1, s1).wait()

            @pl.when(g + 1 < nch2)
            def _():
                gather(j0 + 2, buf0, s0)

            pltpu.sync_copy(
                buf1, out_hbm.at[pl.ds(base + (j0 + 1) * CHUNK, CHUNK)])
            return carry

        lax.fori_loop(0, nch2, body, 0)

    return k(table, idx_pad)


def _sc_scatter_add(gated, idx3, zrows):
    mesh = plsc.VectorSubcoreMesh(core_axis_name="c", subcore_axis_name="s")

    NPAD = 10240

    @functools.partial(
        pl.kernel,
        out_type=jax.ShapeDtypeStruct((2 * N_NODES, EMB), _f32),
        mesh=mesh,
        scratch_types=[pltpu.VMEM_SHARED((NPAD, EMB), _f32),
                       pltpu.VMEM((CPW, CHUNK), jnp.int32),
                       pltpu.VMEM((CHUNK, EMB), _f32),
                       pltpu.VMEM((CHUNK, EMB), _f32),
                       pltpu.SemaphoreType.DMA,
                       pltpu.SemaphoreType.DMA],
    )
    def k(g_hbm, idx_hbm, z_hbm, out_hbm, acc, idx_v, buf0, buf1, s0, s1):
        cid = lax.axis_index("c")
        sid = lax.axis_index("s")
        w = sid * NC + cid
        row0 = sid * 640
        pltpu.sync_copy(z_hbm, acc.at[pl.ds(row0, 640)])
        plsc.subcore_barrier()
        pltpu.sync_copy(idx_hbm.at[w], idx_v)
        base = w * (CPW * CHUNK)
        nch = jnp.minimum(CPW, TOTAL_CHUNKS - w * CPW)
        nch2 = nch // 2

        def load(j, buf, sem):
            return pltpu.async_copy(
                g_hbm.at[pl.ds(base + j * CHUNK, CHUNK)], buf, sem)

        load(0, buf0, s0)

        def body(g, carry):
            j0 = 2 * g
            pltpu.make_async_copy(
                g_hbm.at[pl.ds(base + j0 * CHUNK, CHUNK)], buf0, s0).wait()
            load(j0 + 1, buf1, s1)
            pltpu.sync_copy(buf0, acc.at[idx_v.at[j0]], add=True)
            pltpu.make_async_copy(
                g_hbm.at[pl.ds(base + (j0 + 1) * CHUNK, CHUNK)],
                buf1, s1).wait()

            @pl.when(g + 1 < nch2)
            def _():
                load(j0 + 2, buf0, s0)

            pltpu.sync_copy(buf1, acc.at[idx_v.at[j0 + 1]], add=True)
            return carry

        lax.fori_loop(0, nch2, body, 0)
        plsc.subcore_barrier()

        @pl.when(sid < NS - 1)
        def _():
            pltpu.sync_copy(acc.at[pl.ds(row0, 640)],
                            out_hbm.at[pl.ds(cid * N_NODES + row0, 640)])

        @pl.when(sid == NS - 1)
        def _():
            pltpu.sync_copy(acc.at[pl.ds(row0, 400)],
                            out_hbm.at[pl.ds(cid * N_NODES + row0, 400)])

    return k(gated, idx3, zrows)


def _compose_conv(cp):
    wq, bq = cp["lin_query"]["w"], cp["lin_query"]["b"]
    wk, bk = cp["lin_key"]["w"], cp["lin_key"]["b"]
    wv, bv = cp["lin_value"]["w"], cp["lin_value"]["b"]
    we, be = cp["lin_edge"]["w"], cp["lin_edge"]["b"]
    k0, k0b = cp["key_update"]["l0"]["w"], cp["key_update"]["l0"]["b"]
    k1, k1b = cp["key_update"]["l1"]["w"], cp["key_update"]["l1"]["b"]
    m0, m0b = cp["lin_msg_update"]["l0"]["w"], cp["lin_msg_update"]["l0"]["b"]
    m1, m1b = cp["lin_msg_update"]["l1"]["w"], cp["lin_msg_update"]["l1"]["b"]
    k0i, k0j, k0e = k0[:EMB], k0[EMB:2 * EMB], k0[2 * EMB:]
    m0i, m0j, m0e = m0[:EMB], m0[EMB:2 * EMB], m0[2 * EMB:]
    wd = jnp.concatenate([wq, wk @ k0i, wv @ m0i], axis=1)
    bd = jnp.concatenate(
        [bq, jnp.zeros((EMB,), _f32), jnp.zeros((EMB,), _f32)])[None, :]
    ws = jnp.concatenate([wk @ k0j, wv @ m0j], axis=1)
    wek = we @ k0e
    wem = we @ m0e
    ck = k0b + bk @ k0i + bk @ k0j + be @ k0e
    cm = m0b + bv @ m0i + bv @ m0j + be @ m0e
    cvec = jnp.stack([ck, k1b, cm, m1b], axis=0)
    return dict(wd=wd, bd=bd, ws=ws, wek=wek, wem=wem, k1=k1, m1=m1,
                cvec=cvec, wc=cp["lin_concate"]["w"],
                bc=cp["lin_concate"]["b"][None, :])


def _compose_edge(cp):
    wq, bq = cp["lin_query"]["w"], cp["lin_query"]["b"]
    wk, bk = cp["lin_key"]["w"], cp["lin_key"]["b"]
    wv, bv = cp["lin_value"]["w"], cp["lin_value"]["b"]
    we = cp["lin_edge"]["w"]
    k0, k0b = cp["key_update"]["l0"]["w"], cp["key_update"]["l0"]["b"]
    k1, k1b = cp["key_update"]["l1"]["w"], cp["key_update"]["l1"]["b"]
    m0, m0b = cp["lin_msg_update"]["l0"]["w"], cp["lin_msg_update"]["l0"]["b"]
    m1, m1b = cp["lin_msg_update"]["l1"]["w"], cp["lin_msg_update"]["l1"]["b"]
    k0x, k0y, k0e = k0[:EMB], k0[EMB:2 * EMB], k0[2 * EMB:]
    m0x, m0y, m0e = m0[:EMB], m0[EMB:2 * EMB], m0[2 * EMB:]
    akx = wk @ k0x
    av = wv @ m0x
    ckm = we @ k0e
    cvm = we @ m0e
    bks, bvs, cks, cvs = [], [], [], []
    for t, (ke, ve) in enumerate((("lin_key_e1", "lin_value_e1"),
                                  ("lin_key_e2", "lin_value_e2"),
                                  ("lin_key_e3", "lin_value_e3"))):
        wke, bke = cp[ke]["w"], cp[ke]["b"]
        wve, bve = cp[ve]["w"], cp[ve]["b"]
        bks.append(wke @ k0y)
        bvs.append(wve @ m0y)
        cks.append(k0b + bk @ k0x + bke @ k0y)
        cvs.append(m0b + bv @ m0x + bve @ m0y)
    oth = jnp.stack([bq, k1b, m1b], axis=0)
    return dict(wq=wq, akx=akx, av=av,
                bks=jnp.concatenate(bks, axis=0),
                bvs=jnp.concatenate(bvs, axis=0),
                ck=ckm, cv=cvm, k1=k1, m1=m1,
                cks=jnp.stack(cks, axis=0), cvs=jnp.stack(cvs, axis=0),
                oth=oth, wc=cp["lin_concate"]["w"],
                bc3=3.0 * cp["lin_concate"]["b"][None, :])


def kernel(x, edge_index, edge_attr, edge_nei, batch, params):
    src = edge_index[0]
    dst = edge_index[1]
    comp0 = _compose_conv(params["att0"])
    comp1 = _compose_conv(params["att1"])
    compe = _compose_edge(params["edge_update"])
    wrbf, brbf = params["rbf"]["w"], params["rbf"]["b"][None, :]
    wrba, brba = params["rbf_angle"]["w"], params["rbf_angle"]["b"][None, :]

    pad = EPAD - N_EDGES
    src_pad = jnp.pad(src, (0, pad))
    dst_pad = jnp.pad(dst, (0, pad))
    dst3 = dst_pad.reshape(NW, CPW, CHUNK)
    zrows = jnp.zeros((640, EMB), _f32)
    batch3 = batch.reshape(N_NODES // RN, 1, RN)

    node0, td0, ts0 = _embed_tables(
        x, params["atom_embedding"]["w"],
        params["atom_embedding"]["b"][None, :],
        comp0["wd"], comp0["bd"], comp0["ws"])
    gd0 = _sc_gather(td0, dst_pad, 3 * EMB)
    gs0 = _sc_gather(ts0, src_pad, 2 * EMB)

    ef0 = _rbf_edge(edge_attr, wrbf, brbf)
    nl0, nl1, nl2, na0, na1, na2 = _rbf_nei(
        edge_nei.reshape(N_EDGES, 9), edge_attr,
        wrbf, brbf, wrba, brba)

    alpha0, msg0, asums0 = _conv_phase1(
        gd0, gs0, (ef0,), comp0["wek"], comp0["k1"], comp0["wem"],
        comp0["m1"], comp0["cvec"], pre_bn=False)
    gated0 = _conv_gate(alpha0, msg0, asums0)
    agg0 = _sc_scatter_add(gated0, dst3, zrows)

    a0, a1, a2, v0, v1, v2, easums = _edge_phase_a(
        ef0, (nl0, nl1, nl2), (na0, na1, na2),
        compe["wq"], compe["akx"], compe["av"], compe["bks"],
        compe["bvs"], compe["ck"], compe["cv"], compe["k1"], compe["m1"],
        compe["cks"], compe["cvs"], compe["oth"])

    outlin0, osums0 = _conv_concate(agg0, comp0["wc"], comp0["bc"])
    node1, td1, ts1 = _node_update(node0, outlin0, osums0, comp1["wd"],
                                   comp1["bd"], comp1["ws"],
                                   make_tables=True)
    gd1 = _sc_gather(td1, dst_pad, 3 * EMB)
    gs1 = _sc_gather(ts1, src_pad, 2 * EMB)

    outlin_e, esums = _edge_phase_b((a0, a1, a2), (v0, v1, v2), easums,
                                    compe["wc"], compe["bc3"])

    alpha1, msg1, asums1 = _conv_phase1(
        gd1, gs1, (ef0, outlin_e, esums), comp1["wek"], comp1["k1"],
        comp1["wem"], comp1["m1"], comp1["cvec"], pre_bn=True)
    gated1 = _conv_gate(alpha1, msg1, asums1)
    agg1 = _sc_scatter_add(gated1, dst3, zrows)
    outlin1, osums1 = _conv_concate(agg1, comp1["wc"], comp1["bc"])
    node2 = _node_update(node1, outlin1, osums1, None, None, None,
                         make_tables=False)

    out = _pool_head(node2, batch3,
                     params["fc0"]["w"], params["fc0"]["b"][None, :],
                     params["fc1"]["w"], params["fc1"]["b"][None, :],
                     params["fc_out"]["w"], params["fc_out"]["b"][None, :])
    return jnp.squeeze(out)


# speedup vs baseline: 2.1147x; 1.0017x over previous
import functools
import math

import jax
import jax.numpy as jnp
from jax import lax
from jax.experimental import pallas as pl
from jax.experimental.pallas import tpu as pltpu
from jax.experimental.pallas import tpu_sc as plsc

N_NODES = 10000
N_EDGES = 160000
N_GRAPHS = 64
EMB = 128
BINS = 512

RE = 640
RN = 1000
NC, NS = 2, 16
NW = NC * NS
CHUNK = 128
CPW = 40
TOTAL_CHUNKS = N_EDGES // CHUNK
EPAD = NW * CPW * CHUNK
ROWS_PER_TILE = N_NODES // NS
ISQ = 1.0 / math.sqrt(EMB)

_f32 = jnp.float32


def _tc(body, grid, in_specs, out_specs, out_shape, name):
    return pl.pallas_call(
        body,
        grid=grid,
        in_specs=in_specs,
        out_specs=out_specs,
        out_shape=out_shape,
        compiler_params=pltpu.CompilerParams(
            dimension_semantics=("arbitrary",)),
        name=name,
    )


def _full(shape):
    return pl.BlockSpec(shape, lambda i: (0,) * len(shape))


def _rows(r, cols):
    return pl.BlockSpec((r, cols), lambda i: (i, 0))


def _embed_tables(x, wemb, bemb, wd, bd, ws):
    def body(x_r, wemb_r, bemb_r, wd_r, bd_r, ws_r, node_r, td_r, ts_r):
        node = jnp.dot(x_r[...], wemb_r[...], preferred_element_type=_f32)
        node = node + bemb_r[...]
        node_r[...] = node
        td_r[...] = jnp.dot(node, wd_r[...], preferred_element_type=_f32) + bd_r[...]
        ts_r[...] = jnp.dot(node, ws_r[...], preferred_element_type=_f32)

    return _tc(
        body, (N_NODES // RN,),
        [_rows(RN, 92), _full((92, EMB)), _full((1, EMB)),
         _full((EMB, 3 * EMB)), _full((1, 3 * EMB)), _full((EMB, 2 * EMB))],
        [_rows(RN, EMB), _rows(RN, 3 * EMB), _rows(RN, 2 * EMB)],
        [jax.ShapeDtypeStruct((N_NODES, EMB), _f32),
         jax.ShapeDtypeStruct((N_NODES, 3 * EMB), _f32),
         jax.ShapeDtypeStruct((N_NODES, 2 * EMB), _f32)],
        "embed_tables",
    )(x, wemb, bemb, wd, bd, ws)


def _node_update(node_prev, outlin, sums, wd, bd, ws, make_tables):
    def body(np_r, ol_r, sums_r, wd_r, bd_r, ws_r, node_r, td_r, ts_r):
        s = sums_r[...]
        m = s[0:1, :] / N_NODES
        v = s[1:2, :] / N_NODES - m * m
        node = jax.nn.softplus(np_r[...] + (ol_r[...] - m) / jnp.sqrt(v + 1e-5))
        node_r[...] = node
        td_r[...] = jnp.dot(node, wd_r[...], preferred_element_type=_f32) + bd_r[...]
        ts_r[...] = jnp.dot(node, ws_r[...], preferred_element_type=_f32)

    def body_plain(np_r, ol_r, sums_r, node_r):
        s = sums_r[...]
        m = s[0:1, :] / N_NODES
        v = s[1:2, :] / N_NODES - m * m
        node_r[...] = jax.nn.softplus(
            np_r[...] + (ol_r[...] - m) / jnp.sqrt(v + 1e-5))

    if make_tables:
        return _tc(
            body, (N_NODES // RN,),
            [_rows(RN, EMB), _rows(RN, EMB), _full((2, EMB)),
             _full((EMB, 3 * EMB)), _full((1, 3 * EMB)), _full((EMB, 2 * EMB))],
            [_rows(RN, EMB), _rows(RN, 3 * EMB), _rows(RN, 2 * EMB)],
            [jax.ShapeDtypeStruct((N_NODES, EMB), _f32),
             jax.ShapeDtypeStruct((N_NODES, 3 * EMB), _f32),
             jax.ShapeDtypeStruct((N_NODES, 2 * EMB), _f32)],
            "node_update_tables",
        )(node_prev, outlin, sums, wd, bd, ws)
    return _tc(
        body_plain, (N_NODES // RN,),
        [_rows(RN, EMB), _rows(RN, EMB), _full((2, EMB))],
        [_rows(RN, EMB)],
        [jax.ShapeDtypeStruct((N_NODES, EMB), _f32)],
        "node_update",
    )(node_prev, outlin, sums)[0]


def _rbf_expand(d, vmin, vmax, w, b):
    gamma = (BINS - 1) / (vmax - vmin)
    step = (vmax - vmin) / (BINS - 1)
    kk = -gamma * 1.4426950408889634
    c = vmin + step * lax.broadcasted_iota(
        jnp.int32, (RE, BINS), 1).astype(_f32)
    u = d - c
    ex = jnp.exp2((kk * u) * u)
    return jax.nn.softplus(
        jnp.dot(ex, w, preferred_element_type=_f32) + b)


def _rbf_edge(vecs, w, b):
    L = vecs.shape[0]

    def body(v_r, w_r, b_r, o_r):
        v = v_r[...]
        d = -0.75 / jnp.sqrt(jnp.sum(v * v, axis=1, keepdims=True))
        o_r[...] = _rbf_expand(d, -4.0, 0.0, w_r[...], b_r[...])

    return _tc(
        body, (L // RE,),
        [_rows(RE, 3), _full((BINS, EMB)), _full((1, EMB))],
        [_rows(RE, EMB)],
        [jax.ShapeDtypeStruct((L, EMB), _f32)],
        "rbf_norm",
    )(vecs, w, b)[0]


def _rbf_nei(nei9, att, w_len, b_len, w_ang, b_ang):
    nsteps = N_EDGES // RE

    def body(v_r, a_r, wl_r, bl_r, wa_r, ba_r,
             l0, l1, l2, c0, c1, c2):
        v = v_r[...]
        a = a_r[...]
        vv = v * v
        av = v * jnp.concatenate([a, a, a], axis=1)
        ana = jnp.sqrt(jnp.sum(a * a, axis=1, keepdims=True))
        lane = lax.broadcasted_iota(jnp.int32, (RE, 9), 1)
        louts = (l0, l1, l2)
        couts = (c0, c1, c2)
        for t in range(3):
            m = (lane >= 3 * t) & (lane < 3 * t + 3)
            ss = jnp.sum(jnp.where(m, vv, 0.0), axis=1, keepdims=True)
            nn = jnp.sqrt(ss)
            d = -0.75 / nn
            louts[t][...] = _rbf_expand(d, -4.0, 0.0, wl_r[...], bl_r[...])
            dot = jnp.sum(jnp.where(m, av, 0.0), axis=1, keepdims=True)
            cos = jnp.clip(dot / (nn * ana), -1.0, 1.0)
            couts[t][...] = _rbf_expand(cos, -1.0, 1.0, wa_r[...], ba_r[...])

    e_shape = jax.ShapeDtypeStruct((N_EDGES, EMB), _f32)
    return _tc(
        body, (nsteps,),
        [_rows(RE, 9), _rows(RE, 3), _full((BINS, EMB)),
         _full((1, EMB)), _full((BINS, EMB)), _full((1, EMB))],
        [_rows(RE, EMB)] * 6,
        [e_shape] * 6,
        "rbf_nei",
    )(nei9, att, w_len, b_len, w_ang, b_ang)


def _conv_phase1(gd, gs, ef_args, wek, k1, wem, m1, cvec, pre_bn):
    nsteps = N_EDGES // RE

    def compute(gd_r, gs_r, ef, wek_r, k1_r, wem_r, m1_r, cvec_r,
                alpha_r, msg_r, sums_r):
        c = cvec_r[...]
        g = gd_r[...]
        h = gs_r[...]
        qd = g[:, 0:EMB]
        kad = g[:, EMB:2 * EMB]
        vad = g[:, 2 * EMB:3 * EMB]
        kas = h[:, 0:EMB]
        vas = h[:, EMB:2 * EMB]
        hk = jax.nn.silu(kad + kas +
                         jnp.dot(ef, wek_r[...], preferred_element_type=_f32)
                         + c[0:1, :])
        key_j = jnp.dot(hk, k1_r[...], preferred_element_type=_f32) + c[1:2, :]
        alpha = qd * key_j * ISQ
        hm = jax.nn.silu(vad + vas +
                         jnp.dot(ef, wem_r[...], preferred_element_type=_f32)
                         + c[2:3, :])
        msg = jnp.dot(hm, m1_r[...], preferred_element_type=_f32) + c[3:4, :]
        alpha_r[...] = alpha
        msg_r[...] = msg

        @pl.when(pl.program_id(0) == 0)
        def _():
            sums_r[...] = jnp.zeros_like(sums_r)

        part = jnp.concatenate(
            [jnp.sum(alpha, axis=0, keepdims=True),
             jnp.sum(alpha * alpha, axis=0, keepdims=True)], axis=0)
        sums_r[...] = sums_r[...] + part

    out_specs = [_rows(RE, EMB), _rows(RE, EMB), _full((2, EMB))]
    out_shape = [jax.ShapeDtypeStruct((N_EDGES, EMB), _f32),
                 jax.ShapeDtypeStruct((N_EDGES, EMB), _f32),
                 jax.ShapeDtypeStruct((2, EMB), _f32)]

    if not pre_bn:
        (ef,) = ef_args

        def body(gd_r, gs_r, ef_r, wek_r, k1_r, wem_r, m1_r, cvec_r,
                 alpha_r, msg_r, sums_r):
            compute(gd_r, gs_r, ef_r[...], wek_r, k1_r, wem_r, m1_r, cvec_r,
                    alpha_r, msg_r, sums_r)

        return _tc(
            body, (nsteps,),
            [_rows(RE, 3 * EMB), _rows(RE, 2 * EMB), _rows(RE, EMB),
             _full((EMB, EMB)), _full((EMB, EMB)), _full((EMB, EMB)),
             _full((EMB, EMB)), _full((4, EMB))],
            out_specs, out_shape, "conv_phase1",
        )(gd, gs, ef, wek, k1, wem, m1, cvec)

    ef0, outlin_e, esums = ef_args

    def body2(gd_r, gs_r, ef0_r, ol_r, es_r, wek_r, k1_r, wem_r, m1_r,
              cvec_r, alpha_r, msg_r, sums_r):
        s = es_r[...]
        m = s[0:1, :] / N_EDGES
        v = s[1:2, :] / N_EDGES - m * m
        ef = jax.nn.softplus(ef0_r[...] + (ol_r[...] - m) / jnp.sqrt(v + 1e-5))
        compute(gd_r, gs_r, ef, wek_r, k1_r, wem_r, m1_r, cvec_r,
                alpha_r, msg_r, sums_r)

    return _tc(
        body2, (nsteps,),
        [_rows(RE, 3 * EMB), _rows(RE, 2 * EMB), _rows(RE, EMB),
         _rows(RE, EMB), _full((2, EMB)),
         _full((EMB, EMB)), _full((EMB, EMB)), _full((EMB, EMB)),
         _full((EMB, EMB)), _full((4, EMB))],
        out_specs, out_shape, "conv_phase1_bn",
    )(gd, gs, ef0, outlin_e, esums, wek, k1, wem, m1, cvec)


def _conv_gate(alpha, msg, sums):
    def body(a_r, m_r, s_r, o_r):
        s = s_r[...]
        mean = s[0:1, :] / N_EDGES
        var = s[1:2, :] / N_EDGES - mean * mean
        bn = (a_r[...] - mean) / jnp.sqrt(var + 1e-5)
        o_r[...] = m_r[...] * jax.nn.sigmoid(bn)

    return _tc(
        body, (N_EDGES // RE,),
        [_rows(RE, EMB), _rows(RE, EMB), _full((2, EMB))],
        [_rows(RE, EMB)],
        [jax.ShapeDtypeStruct((N_EDGES, EMB), _f32)],
        "conv_gate",
    )(alpha, msg, sums)[0]


def _conv_concate(agg2, wc, bc):
    nsteps = N_NODES // RN

    def body(a_r, b_r, wc_r, bc_r, o_r, sums_r):
        s = a_r[...] + b_r[...]
        o = jnp.dot(s, wc_r[...], preferred_element_type=_f32) + bc_r[...]
        o_r[...] = o

        @pl.when(pl.program_id(0) == 0)
        def _():
            sums_r[...] = jnp.zeros_like(sums_r)

        part = jnp.concatenate(
            [jnp.sum(o, axis=0, keepdims=True),
             jnp.sum(o * o, axis=0, keepdims=True)], axis=0)
        sums_r[...] = sums_r[...] + part

    spec_a = pl.BlockSpec((RN, EMB), lambda i: (i, 0))
    spec_b = pl.BlockSpec((RN, EMB), lambda i: (i + N_NODES // RN, 0))
    return _tc(
        body, (nsteps,),
        [spec_a, spec_b, _full((EMB, EMB)), _full((1, EMB))],
        [_rows(RN, EMB), _full((2, EMB))],
        [jax.ShapeDtypeStruct((N_NODES, EMB), _f32),
         jax.ShapeDtypeStruct((2, EMB), _f32)],
        "conv_concate",
    )(agg2, agg2, wc, bc)


def _edge_phase_a(ef, nls, nas, wq, akx, av, bks, bvs, ck, cv, k1, m1,
                  cks, cvs, oth):
    nsteps = N_EDGES // RE

    def body(ef_r, nl0, nl1, nl2, na0, na1, na2, wq_r, akx_r, av_r,
             bks_r, bvs_r, ck_r, cv_r, k1_r, m1_r, cks_r, cvs_r, oth_r,
             a0, a1, a2, v0, v1, v2, sums_r):
        e = ef_r[...]
        othv = oth_r[...]
        q = jnp.dot(e, wq_r[...], preferred_element_type=_f32) + othv[0:1, :]
        ekx = jnp.dot(e, akx_r[...], preferred_element_type=_f32)
        evx = jnp.dot(e, av_r[...], preferred_element_type=_f32)
        k1v = k1_r[...]
        m1v = m1_r[...]
        ckv = ck_r[...]
        cvv = cv_r[...]
        bksv = bks_r[...]
        bvsv = bvs_r[...]
        cksv = cks_r[...]
        cvsv = cvs_r[...]
        nls = (nl0[...], nl1[...], nl2[...])
        nas = (na0[...], na1[...], na2[...])
        aouts = (a0, a1, a2)
        vouts = (v0, v1, v2)
        ssum = jnp.zeros((1, EMB), _f32)
        ssq = jnp.zeros((1, EMB), _f32)
        for t in range(3):
            bk_t = bksv[t * EMB:(t + 1) * EMB, :]
            bv_t = bvsv[t * EMB:(t + 1) * EMB, :]
            hk = jax.nn.silu(
                ekx + jnp.dot(nls[t], bk_t, preferred_element_type=_f32)
                + jnp.dot(nas[t], ckv, preferred_element_type=_f32)
                + cksv[t:t + 1, :])
            kt = jnp.dot(hk, k1v, preferred_element_type=_f32) + othv[1:2, :]
            at = q * kt * ISQ
            aouts[t][...] = at
            hv = jax.nn.silu(
                evx + jnp.dot(nls[t], bv_t, preferred_element_type=_f32)
                + jnp.dot(nas[t], cvv, preferred_element_type=_f32)
                + cvsv[t:t + 1, :])
            vt = jnp.dot(hv, m1v, preferred_element_type=_f32) + othv[2:3, :]
            vouts[t][...] = vt
            ssum = ssum + jnp.sum(at, axis=0, keepdims=True)
            ssq = ssq + jnp.sum(at * at, axis=0, keepdims=True)

        @pl.when(pl.program_id(0) == 0)
        def _():
            sums_r[...] = jnp.zeros_like(sums_r)

        sums_r[...] = sums_r[...] + jnp.concatenate([ssum, ssq], axis=0)

    e_shape = jax.ShapeDtypeStruct((N_EDGES, EMB), _f32)
    return _tc(
        body, (nsteps,),
        [_rows(RE, EMB)] * 7 +
        [_full((EMB, EMB)), _full((EMB, EMB)), _full((EMB, EMB)),
         _full((3 * EMB, EMB)), _full((3 * EMB, EMB)),
         _full((EMB, EMB)), _full((EMB, EMB)),
         _full((EMB, EMB)), _full((EMB, EMB)),
         _full((3, EMB)), _full((3, EMB)), _full((3, EMB))],
        [_rows(RE, EMB)] * 6 + [_full((2, EMB))],
        [e_shape] * 6 + [jax.ShapeDtypeStruct((2, EMB), _f32)],
        "edge_phase_a",
    )(ef, *nls, *nas, wq, akx, av, bks, bvs, ck, cv, k1, m1,
      cks, cvs, oth)


def _edge_phase_b(alphas, vals, asums, wc, bc3):
    nsteps = N_EDGES // RE

    def body(a0, a1, a2, v0, v1, v2, as_r, wc_r, bc3_r, o_r, sums_r):
        s = as_r[...]
        mean = s[0:1, :] / (3 * N_EDGES)
        var = s[1:2, :] / (3 * N_EDGES) - mean * mean
        rstd = 1.0 / jnp.sqrt(var + 1e-5)
        acc = jnp.zeros((RE, EMB), _f32)
        for a_r, v_r in ((a0, v0), (a1, v1), (a2, v2)):
            gate = jax.nn.sigmoid((a_r[...] - mean) * rstd)
            acc = acc + v_r[...] * gate
        o = jnp.dot(acc, wc_r[...], preferred_element_type=_f32) + bc3_r[...]
        o_r[...] = o

        @pl.when(pl.program_id(0) == 0)
        def _():
            sums_r[...] = jnp.zeros_like(sums_r)

        part = jnp.concatenate(
            [jnp.sum(o, axis=0, keepdims=True),
             jnp.sum(o * o, axis=0, keepdims=True)], axis=0)
        sums_r[...] = sums_r[...] + part

    return _tc(
        body, (nsteps,),
        [_rows(RE, EMB)] * 6 + [_full((2, EMB)), _full((EMB, EMB)),
                                _full((1, EMB))],
        [_rows(RE, EMB), _full((2, EMB))],
        [jax.ShapeDtypeStruct((N_EDGES, EMB), _f32),
         jax.ShapeDtypeStruct((2, EMB), _f32)],
        "edge_phase_b",
    )(*alphas, *vals, asums, wc, bc3)


def _pool_head(node, batch3, w0, b0, w1, b1, w2, b2):
    nsteps = N_NODES // RN

    def body(n_r, bt_r, w0_r, b0_r, w1_r, b1_r, w2_r, b2_r, o_r,
             sacc, cacc):
        @pl.when(pl.program_id(0) == 0)
        def _():
            sacc[...] = jnp.zeros_like(sacc)
            cacc[...] = jnp.zeros_like(cacc)

        b = bt_r[0]
        oh = (lax.broadcasted_iota(jnp.int32, (N_GRAPHS, RN), 0)
              == b).astype(_f32)
        sacc[...] = sacc[...] + lax.dot_general(
            oh, n_r[...], (((1,), (0,)), ((), ())),
            preferred_element_type=_f32)
        cacc[...] = cacc[...] + jnp.broadcast_to(
            jnp.sum(oh, axis=1, keepdims=True), (N_GRAPHS, EMB))

        @pl.when(pl.program_id(0) == nsteps - 1)
        def _():
            feats = sacc[...] / jnp.maximum(cacc[...], 1.0)
            h = jax.nn.silu(
                jnp.dot(feats, w0_r[...], preferred_element_type=_f32)
                + b0_r[...])
            h = jax.nn.silu(
                jnp.dot(h, w1_r[...], preferred_element_type=_f32)
                + b1_r[...])
            o_r[...] = (jnp.dot(h, w2_r[...], preferred_element_type=_f32)
                        + b2_r[...])

    return pl.pallas_call(
        body,
        grid=(nsteps,),
        in_specs=[_rows(RN, EMB),
                  pl.BlockSpec((1, 1, RN), lambda i: (i, 0, 0)),
                  _full((EMB, EMB)), _full((1, EMB)),
                  _full((EMB, EMB)), _full((1, EMB)),
                  _full((EMB, 6)), _full((1, 6))],
        out_specs=_full((N_GRAPHS, 6)),
        out_shape=jax.ShapeDtypeStruct((N_GRAPHS, 6), _f32),
        scratch_shapes=[pltpu.VMEM((N_GRAPHS, EMB), _f32),
                        pltpu.VMEM((N_GRAPHS, EMB), _f32)],
        compiler_params=pltpu.CompilerParams(
            dimension_semantics=("arbitrary",)),
        name="pool_head",
    )(node, batch3, w0, b0, w1, b1, w2, b2)


def _sc_gather(table, idx_pad, width):
    mesh = plsc.VectorSubcoreMesh(core_axis_name="c", subcore_axis_name="s")

    @functools.partial(
        pl.kernel,
        out_type=jax.ShapeDtypeStruct((N_EDGES, width), _f32),
        mesh=mesh,
        scratch_types=[pltpu.VMEM((CPW * CHUNK,), jnp.int32),
                       pltpu.VMEM((CHUNK, width), _f32),
                       pltpu.VMEM((CHUNK, width), _f32),
                       pltpu.SemaphoreType.DMA,
                       pltpu.SemaphoreType.DMA,
                       pltpu.SemaphoreType.DMA,
                       pltpu.SemaphoreType.DMA],
    )
    def k(table_hbm, idx_hbm, out_hbm, idx_v, buf0, buf1, s0, s1, o0, o1):
        w = lax.axis_index("s") * NC + lax.axis_index("c")
        base = w * (CPW * CHUNK)
        pltpu.sync_copy(idx_hbm.at[pl.ds(base, CPW * CHUNK)], idx_v)
        nch = jnp.minimum(CPW, TOTAL_CHUNKS - w * CPW)
        nch2 = nch // 2

        def gdesc(j, buf, sem):
            return pltpu.make_async_copy(
                table_hbm.at[idx_v.at[pl.ds(j * CHUNK, CHUNK)]], buf, sem)

        def odesc(j, buf, sem):
            return pltpu.make_async_copy(
                buf, out_hbm.at[pl.ds(base + j * CHUNK, CHUNK)], sem)

        pltpu.async_copy(
            table_hbm.at[idx_v.at[pl.ds(0, CHUNK)]], buf0, s0)
        pltpu.async_copy(
            table_hbm.at[idx_v.at[pl.ds(CHUNK, CHUNK)]], buf1, s1)

        def body(g, carry):
            j0 = 2 * g
            gdesc(j0, buf0, s0).wait()
            pltpu.async_copy(buf0,
                             out_hbm.at[pl.ds(base + j0 * CHUNK, CHUNK)],
                             o0)
            gdesc(j0 + 1, buf1, s1).wait()
            pltpu.async_copy(
                buf1, out_hbm.at[pl.ds(base + (j0 + 1) * CHUNK, CHUNK)],
                o1)
            odesc(j0, buf0, o0).wait()
            odesc(j0 + 1, buf1, o1).wait()

            @pl.when(g + 1 < nch2)
            def _():
                pltpu.async_copy(
                    table_hbm.at[idx_v.at[pl.ds((j0 + 2) * CHUNK, CHUNK)]],
                    buf0, s0)
                pltpu.async_copy(
                    table_hbm.at[idx_v.at[pl.ds((j0 + 3) * CHUNK, CHUNK)]],
                    buf1, s1)

            return carry

        lax.fori_loop(0, nch2, body, 0)

    return k(table, idx_pad)


def _sc_scatter_add(gated, idx3, zrows):
    mesh = plsc.VectorSubcoreMesh(core_axis_name="c", subcore_axis_name="s")

    NPAD = 10240

    @functools.partial(
        pl.kernel,
        out_type=jax.ShapeDtypeStruct((2 * N_NODES, EMB), _f32),
        mesh=mesh,
        scratch_types=[pltpu.VMEM_SHARED((NPAD, EMB), _f32),
                       pltpu.VMEM((CPW, CHUNK), jnp.int32),
                       pltpu.VMEM((CHUNK, EMB), _f32),
                       pltpu.VMEM((CHUNK, EMB), _f32),
                       pltpu.SemaphoreType.DMA,
                       pltpu.SemaphoreType.DMA],
    )
    def k(g_hbm, idx_hbm, z_hbm, out_hbm, acc, idx_v, buf0, buf1, s0, s1):
        cid = lax.axis_index("c")
        sid = lax.axis_index("s")
        w = sid * NC + cid
        row0 = sid * 640
        pltpu.sync_copy(z_hbm, acc.at[pl.ds(row0, 640)])
        plsc.subcore_barrier()
        pltpu.sync_copy(idx_hbm.at[w], idx_v)
        base = w * (CPW * CHUNK)
        nch = jnp.minimum(CPW, TOTAL_CHUNKS - w * CPW)
        nch2 = nch // 2

        def load(j, buf, sem):
            return pltpu.async_copy(
                g_hbm.at[pl.ds(base + j * CHUNK, CHUNK)], buf, sem)

        load(0, buf0, s0)

        def body(g, carry):
            j0 = 2 * g
            pltpu.make_async_copy(
                g_hbm.at[pl.ds(base + j0 * CHUNK, CHUNK)], buf0, s0).wait()
            load(j0 + 1, buf1, s1)
            pltpu.sync_copy(buf0, acc.at[idx_v.at[j0]], add=True)
            pltpu.make_async_copy(
                g_hbm.at[pl.ds(base + (j0 + 1) * CHUNK, CHUNK)],
                buf1, s1).wait()

            @pl.when(g + 1 < nch2)
            def _():
                load(j0 + 2, buf0, s0)

            pltpu.sync_copy(buf1, acc.at[idx_v.at[j0 + 1]], add=True)
            return carry

        lax.fori_loop(0, nch2, body, 0)
        plsc.subcore_barrier()

        @pl.when(sid < NS - 1)
        def _():
            pltpu.sync_copy(acc.at[pl.ds(row0, 640)],
                            out_hbm.at[pl.ds(cid * N_NODES + row0, 640)])

        @pl.when(sid == NS - 1)
        def _():
            pltpu.sync_copy(acc.at[pl.ds(row0, 400)],
                            out_hbm.at[pl.ds(cid * N_NODES + row0, 400)])

    return k(gated, idx3, zrows)


def _compose_conv(cp):
    wq, bq = cp["lin_query"]["w"], cp["lin_query"]["b"]
    wk, bk = cp["lin_key"]["w"], cp["lin_key"]["b"]
    wv, bv = cp["lin_value"]["w"], cp["lin_value"]["b"]
    we, be = cp["lin_edge"]["w"], cp["lin_edge"]["b"]
    k0, k0b = cp["key_update"]["l0"]["w"], cp["key_update"]["l0"]["b"]
    k1, k1b = cp["key_update"]["l1"]["w"], cp["key_update"]["l1"]["b"]
    m0, m0b = cp["lin_msg_update"]["l0"]["w"], cp["lin_msg_update"]["l0"]["b"]
    m1, m1b = cp["lin_msg_update"]["l1"]["w"], cp["lin_msg_update"]["l1"]["b"]
    k0i, k0j, k0e = k0[:EMB], k0[EMB:2 * EMB], k0[2 * EMB:]
    m0i, m0j, m0e = m0[:EMB], m0[EMB:2 * EMB], m0[2 * EMB:]
    wd = jnp.concatenate([wq, wk @ k0i, wv @ m0i], axis=1)
    bd = jnp.concatenate(
        [bq, jnp.zeros((EMB,), _f32), jnp.zeros((EMB,), _f32)])[None, :]
    ws = jnp.concatenate([wk @ k0j, wv @ m0j], axis=1)
    wek = we @ k0e
    wem = we @ m0e
    ck = k0b + bk @ k0i + bk @ k0j + be @ k0e
    cm = m0b + bv @ m0i + bv @ m0j + be @ m0e
    cvec = jnp.stack([ck, k1b, cm, m1b], axis=0)
    return dict(wd=wd, bd=bd, ws=ws, wek=wek, wem=wem, k1=k1, m1=m1,
                cvec=cvec, wc=cp["lin_concate"]["w"],
                bc=cp["lin_concate"]["b"][None, :])


def _compose_edge(cp):
    wq, bq = cp["lin_query"]["w"], cp["lin_query"]["b"]
    wk, bk = cp["lin_key"]["w"], cp["lin_key"]["b"]
    wv, bv = cp["lin_value"]["w"], cp["lin_value"]["b"]
    we = cp["lin_edge"]["w"]
    k0, k0b = cp["key_update"]["l0"]["w"], cp["key_update"]["l0"]["b"]
    k1, k1b = cp["key_update"]["l1"]["w"], cp["key_update"]["l1"]["b"]
    m0, m0b = cp["lin_msg_update"]["l0"]["w"], cp["lin_msg_update"]["l0"]["b"]
    m1, m1b = cp["lin_msg_update"]["l1"]["w"], cp["lin_msg_update"]["l1"]["b"]
    k0x, k0y, k0e = k0[:EMB], k0[EMB:2 * EMB], k0[2 * EMB:]
    m0x, m0y, m0e = m0[:EMB], m0[EMB:2 * EMB], m0[2 * EMB:]
    akx = wk @ k0x
    av = wv @ m0x
    ckm = we @ k0e
    cvm = we @ m0e
    bks, bvs, cks, cvs = [], [], [], []
    for t, (ke, ve) in enumerate((("lin_key_e1", "lin_value_e1"),
                                  ("lin_key_e2", "lin_value_e2"),
                                  ("lin_key_e3", "lin_value_e3"))):
        wke, bke = cp[ke]["w"], cp[ke]["b"]
        wve, bve = cp[ve]["w"], cp[ve]["b"]
        bks.append(wke @ k0y)
        bvs.append(wve @ m0y)
        cks.append(k0b + bk @ k0x + bke @ k0y)
        cvs.append(m0b + bv @ m0x + bve @ m0y)
    oth = jnp.stack([bq, k1b, m1b], axis=0)
    return dict(wq=wq, akx=akx, av=av,
                bks=jnp.concatenate(bks, axis=0),
                bvs=jnp.concatenate(bvs, axis=0),
                ck=ckm, cv=cvm, k1=k1, m1=m1,
                cks=jnp.stack(cks, axis=0), cvs=jnp.stack(cvs, axis=0),
                oth=oth, wc=cp["lin_concate"]["w"],
                bc3=3.0 * cp["lin_concate"]["b"][None, :])


def kernel(x, edge_index, edge_attr, edge_nei, batch, params):
    src = edge_index[0]
    dst = edge_index[1]
    comp0 = _compose_conv(params["att0"])
    comp1 = _compose_conv(params["att1"])
    compe = _compose_edge(params["edge_update"])
    wrbf, brbf = params["rbf"]["w"], params["rbf"]["b"][None, :]
    wrba, brba = params["rbf_angle"]["w"], params["rbf_angle"]["b"][None, :]

    pad = EPAD - N_EDGES
    src_pad = jnp.pad(src, (0, pad))
    dst_pad = jnp.pad(dst, (0, pad))
    dst3 = dst_pad.reshape(NW, CPW, CHUNK)
    zrows = jnp.zeros((640, EMB), _f32)
    batch3 = batch.reshape(N_NODES // RN, 1, RN)

    node0, td0, ts0 = _embed_tables(
        x, params["atom_embedding"]["w"],
        params["atom_embedding"]["b"][None, :],
        comp0["wd"], comp0["bd"], comp0["ws"])
    gd0 = _sc_gather(td0, dst_pad, 3 * EMB)
    gs0 = _sc_gather(ts0, src_pad, 2 * EMB)

    ef0 = _rbf_edge(edge_attr, wrbf, brbf)
    nl0, nl1, nl2, na0, na1, na2 = _rbf_nei(
        edge_nei.reshape(N_EDGES, 9), edge_attr,
        wrbf, brbf, wrba, brba)

    alpha0, msg0, asums0 = _conv_phase1(
        gd0, gs0, (ef0,), comp0["wek"], comp0["k1"], comp0["wem"],
        comp0["m1"], comp0["cvec"], pre_bn=False)
    gated0 = _conv_gate(alpha0, msg0, asums0)
    agg0 = _sc_scatter_add(gated0, dst3, zrows)

    a0, a1, a2, v0, v1, v2, easums = _edge_phase_a(
        ef0, (nl0, nl1, nl2), (na0, na1, na2),
        compe["wq"], compe["akx"], compe["av"], compe["bks"],
        compe["bvs"], compe["ck"], compe["cv"], compe["k1"], compe["m1"],
        compe["cks"], compe["cvs"], compe["oth"])

    outlin0, osums0 = _conv_concate(agg0, comp0["wc"], comp0["bc"])
    node1, td1, ts1 = _node_update(node0, outlin0, osums0, comp1["wd"],
                                   comp1["bd"], comp1["ws"],
                                   make_tables=True)
    gd1 = _sc_gather(td1, dst_pad, 3 * EMB)
    gs1 = _sc_gather(ts1, src_pad, 2 * EMB)

    outlin_e, esums = _edge_phase_b((a0, a1, a2), (v0, v1, v2), easums,
                                    compe["wc"], compe["bc3"])

    alpha1, msg1, asums1 = _conv_phase1(
        gd1, gs1, (ef0, outlin_e, esums), comp1["wek"], comp1["k1"],
        comp1["wem"], comp1["m1"], comp1["cvec"], pre_bn=True)
    gated1 = _conv_gate(alpha1, msg1, asums1)
    agg1 = _sc_scatter_add(gated1, dst3, zrows)
    outlin1, osums1 = _conv_concate(agg1, comp1["wc"], comp1["bc"])
    node2 = _node_update(node1, outlin1, osums1, None, None, None,
                         make_tables=False)

    out = _pool_head(node2, batch3,
                     params["fc0"]["w"], params["fc0"]["b"][None, :],
                     params["fc1"]["w"], params["fc1"]["b"][None, :],
                     params["fc_out"]["w"], params["fc_out"]["b"][None, :])
    return jnp.squeeze(out)
